# Initial kernel scaffold; baseline (speedup 1.0000x reference)
import jax
import jax.numpy as jnp
from jax.experimental import pallas as pl


def kernel(z, edge_index, edge_weight, emb, dW1, db1, dW2, db2, sW1, sb1, sW2, sb2, tW1, tb1, tW2, tb2, eW1, eb1, eW2, eb2):
    raise NotImplementedError("write your pallas kernel here")


# speedup vs baseline: 2.8467x; 2.8467x over previous
import functools

import jax
import jax.numpy as jnp
from jax import lax
from jax.experimental import pallas as pl
from jax.experimental.pallas import tpu as pltpu
from jax.experimental.pallas import tpu_sc as plsc

N = 10000
E = 320000
NB = 20
CUTOFF = 5.0

NC = 2
NS = 16
NW = NC * NS
EPW = E // NW
CHUNK = 125
NCHUNK = EPW // CHUNK
ROWS_PER_TILE = N // NS

EB = 2560
NBLK = E // EB


def _silu(x):
    return x * jax.nn.sigmoid(x)


def _sc_gather_z(z, row, col):
    mesh = plsc.VectorSubcoreMesh(core_axis_name="c", subcore_axis_name="s")

    @functools.partial(
        pl.kernel,
        out_type=[jax.ShapeDtypeStruct((E,), jnp.int32),
                  jax.ShapeDtypeStruct((E,), jnp.int32)],
        mesh=mesh,
        scratch_types=[pltpu.VMEM((N,), jnp.int32),
                       pltpu.VMEM((EPW,), jnp.int32),
                       pltpu.VMEM((EPW,), jnp.int32)],
        compiler_params=pltpu.CompilerParams(needs_layout_passes=False),
    )
    def k(z_hbm, row_hbm, col_hbm, zr_hbm, zc_hbm, z_v, idx_v, out_v):
        wid = lax.axis_index("s") * NC + lax.axis_index("c")
        base = wid * EPW
        pltpu.sync_copy(z_hbm, z_v)

        def one(src_hbm, dst_hbm):
            pltpu.sync_copy(src_hbm.at[pl.ds(base, EPW)], idx_v)

            def body(i, _):
                iv = idx_v[pl.ds(i * 16, 16)]
                out_v[pl.ds(i * 16, 16)] = plsc.load_gather(z_v, [iv])
                return ()

            lax.fori_loop(0, EPW // 16, body, (), unroll=4)
            pltpu.sync_copy(out_v, dst_hbm.at[pl.ds(base, EPW)])

        one(row_hbm, zr_hbm)
        one(col_hbm, zc_hbm)

    return k(z, row, col)


def _tc_tables(embp, sW1, sb1, sW2, sb2, tW1, tb1, tW2, tb2,
               eW1_top, eW1_mid, eW1_bot, dW2, db2, eb1):
    def k(embp_r, sW1_r, sb1_r, sW2_r, sb2_r, tW1_r, tb1_r, tW2_r, tb2_r,
          eW1t_r, eW1m_r, eW1b_r, dW2_r, db2_r, eb1_r,
          sc_o, tc_o, m_o, c_o):
        f32 = jnp.float32
        emb_v = embp_r[...]
        s_all = _silu(jnp.dot(emb_v, sW1_r[...], preferred_element_type=f32) + sb1_r[...])
        s_all = jnp.dot(s_all, sW2_r[...], preferred_element_type=f32) + sb2_r[...]
        sc_o[...] = jnp.dot(s_all, eW1m_r[...], preferred_element_type=f32)
        t_all = _silu(jnp.dot(emb_v, tW1_r[...], preferred_element_type=f32) + tb1_r[...])
        t_all = jnp.dot(t_all, tW2_r[...], preferred_element_type=f32) + tb2_r[...]
        tc_o[...] = jnp.dot(t_all, eW1b_r[...], preferred_element_type=f32)
        m_o[...] = jnp.dot(dW2_r[...], eW1t_r[...], preferred_element_type=f32)
        c_o[...] = eb1_r[...] + jnp.dot(db2_r[...], eW1t_r[...], preferred_element_type=f32)

    return pl.pallas_call(
        k,
        out_shape=[jax.ShapeDtypeStruct((128, 128), jnp.float32),
                   jax.ShapeDtypeStruct((128, 128), jnp.float32),
                   jax.ShapeDtypeStruct((128, 128), jnp.float32),
                   jax.ShapeDtypeStruct((1, 128), jnp.float32)],
    )(embp, sW1, sb1.reshape(1, 128), sW2, sb2.reshape(1, 128),
      tW1, tb1.reshape(1, 128), tW2, tb2.reshape(1, 128),
      eW1_top, eW1_mid, eW1_bot, dW2, db2.reshape(1, 128), eb1.reshape(1, 128))


def _tc_edges(w3, zr3, zc3, dW1p, db1, s_con, t_con, M, cvec, eW2, eb2):
    def k(w_r, zr_r, zc_r, dW1_r, db1_r, sc_r, tc_r, m_r, c_r, eW2_r, eb2_r, out_r):
        f32 = jnp.float32
        r = w_r[0, 0, :].reshape(EB, 1)
        n = lax.broadcasted_iota(jnp.int32, (EB, 32), 1).astype(f32) + 1.0
        basis = (jnp.sqrt(2.0 / CUTOFF) / r) * jnp.sin(n * (jnp.pi / CUTOFF) * r)
        h = _silu(jnp.dot(basis, dW1_r[...], preferred_element_type=f32) + db1_r[...])
        pre = jnp.dot(h, m_r[...], preferred_element_type=f32)
        lanes = lax.broadcasted_iota(jnp.int32, (EB, 128), 1)
        ohr = (lanes == zr_r[0, 0, :].reshape(EB, 1)).astype(f32)
        ohc = (lanes == zc_r[0, 0, :].reshape(EB, 1)).astype(f32)
        pre = pre + jnp.dot(ohr, sc_r[...], preferred_element_type=f32)
        pre = pre + jnp.dot(ohc, tc_r[...], preferred_element_type=f32)
        pre = _silu(pre + c_r[...])
        out_r[...] = jnp.dot(pre, eW2_r[...], preferred_element_type=f32) + eb2_r[...]

    full = lambda s: pl.BlockSpec(s, lambda i: (0,) * len(s))
    return pl.pallas_call(
        k,
        grid=(NBLK,),
        in_specs=[
            pl.BlockSpec((1, 1, EB), lambda i: (i, 0, 0)),
            pl.BlockSpec((1, 1, EB), lambda i: (i, 0, 0)),
            pl.BlockSpec((1, 1, EB), lambda i: (i, 0, 0)),
            full((32, 128)), full((1, 128)), full((128, 128)),
            full((128, 128)), full((128, 128)), full((1, 128)),
            full((128, 128)), full((1, 128)),
        ],
        out_specs=pl.BlockSpec((EB, 128), lambda i: (i, 0)),
        out_shape=jax.ShapeDtypeStruct((E, 128), jnp.float32),
    )(w3, zr3, zc3, dW1p, db1, s_con, t_con, M, cvec, eW2, eb2)


def _sc_scatter(ef4, row3, zeros):
    mesh = plsc.VectorSubcoreMesh(core_axis_name="c", subcore_axis_name="s")

    @functools.partial(
        pl.kernel,
        out_type=jax.ShapeDtypeStruct((NC, N, 128), jnp.float32),
        mesh=mesh,
        scratch_types=[pltpu.VMEM((NCHUNK, CHUNK), jnp.int32),
                       pltpu.VMEM((CHUNK, 128), jnp.float32),
                       pltpu.VMEM_SHARED((N, 128), jnp.float32)],
    )
    def k(ef_hbm, row_hbm, zero_hbm, out_hbm, idx_v, buf_v, acc_sh):
        c = lax.axis_index("c")
        s = lax.axis_index("s")
        wid = s * NC + c

        @pl.when(s < 10)
        def _zero():
            pltpu.sync_copy(zero_hbm, acc_sh.at[pl.ds(s * 1000, 1000)])

        plsc.subcore_barrier()
        pltpu.sync_copy(row_hbm.at[wid], idx_v)

        def body(j, _):
            pltpu.sync_copy(ef_hbm.at[wid, j], buf_v)
            pltpu.sync_copy(buf_v, acc_sh.at[idx_v.at[j]], add=True)
            return ()

        lax.fori_loop(0, NCHUNK, body, ())
        plsc.subcore_barrier()

        @pl.when(s < 10)
        def _dump():
            pltpu.sync_copy(acc_sh.at[pl.ds(s * 1000, 1000)],
                            out_hbm.at[c, pl.ds(s * 1000, 1000)])

    return k(ef4, row3, zeros)


def _tc_combine(partials):
    def k(p_r, out_r):
        out_r[...] = p_r[0] + p_r[1]

    return pl.pallas_call(
        k,
        grid=(10,),
        in_specs=[pl.BlockSpec((2, N // 10, 128), lambda i: (0, i, 0))],
        out_specs=pl.BlockSpec((N // 10, 128), lambda i: (i, 0)),
        out_shape=jax.ShapeDtypeStruct((N, 128), jnp.float32),
    )(partials)


def kernel(z, edge_index, edge_weight, emb, dW1, db1, dW2, db2, sW1, sb1,
           sW2, sb2, tW1, tb1, tW2, tb2, eW1, eb1, eW2, eb2):
    z = z.astype(jnp.int32)
    row = edge_index[0].astype(jnp.int32)
    col = edge_index[1].astype(jnp.int32)

    zr, zc = _sc_gather_z(z, row, col)

    embp = jnp.zeros((128, 128), jnp.float32).at[:120].set(emb)
    s_con, t_con, M, cvec = _tc_tables(
        embp, sW1, sb1, sW2, sb2, tW1, tb1, tW2, tb2,
        eW1[0:128], eW1[128:256], eW1[256:384], dW2, db2, eb1)

    dW1p = jnp.zeros((32, 128), jnp.float32).at[:NB].set(dW1)
    ef = _tc_edges(edge_weight.reshape(NBLK, 1, EB),
                   zr.reshape(NBLK, 1, EB), zc.reshape(NBLK, 1, EB),
                   dW1p, db1.reshape(1, 128), s_con, t_con, M, cvec,
                   eW2, eb2.reshape(1, 128))

    partials = _sc_scatter(ef.reshape(NW, NCHUNK, CHUNK, 128),
                           row.reshape(NW, NCHUNK, CHUNK),
                           jnp.zeros((1000, 128), jnp.float32))
    return _tc_combine(partials)


# speedup vs baseline: 4.6109x; 1.6198x over previous
import functools

import jax
import jax.numpy as jnp
from jax import lax
from jax.experimental import pallas as pl
from jax.experimental.pallas import tpu as pltpu
from jax.experimental.pallas import tpu_sc as plsc

N = 10000
E = 320000
NB = 20
CUTOFF = 5.0

NC = 2
NS = 16
NW = NC * NS
EPW = E // NW
CHUNK = 125
NCHUNK = EPW // CHUNK
ROWS_PER_TILE = N // NS

EB = 2560
NBLK = E // EB


def _silu(x):
    return x * jax.nn.sigmoid(x)


def _sc_gather_z(z, row, col):
    mesh = plsc.VectorSubcoreMesh(core_axis_name="c", subcore_axis_name="s")

    @functools.partial(
        pl.kernel,
        out_type=[jax.ShapeDtypeStruct((E,), jnp.int32),
                  jax.ShapeDtypeStruct((E,), jnp.int32)],
        mesh=mesh,
        scratch_types=[pltpu.VMEM((N,), jnp.int32),
                       pltpu.VMEM((EPW,), jnp.int32),
                       pltpu.VMEM((EPW,), jnp.int32)],
        compiler_params=pltpu.CompilerParams(needs_layout_passes=False),
    )
    def k(z_hbm, row_hbm, col_hbm, zr_hbm, zc_hbm, z_v, idx_v, out_v):
        wid = lax.axis_index("s") * NC + lax.axis_index("c")
        base = wid * EPW
        pltpu.sync_copy(z_hbm, z_v)

        def one(src_hbm, dst_hbm):
            pltpu.sync_copy(src_hbm.at[pl.ds(base, EPW)], idx_v)

            def body(i, _):
                iv = idx_v[pl.ds(i * 16, 16)]
                out_v[pl.ds(i * 16, 16)] = plsc.load_gather(z_v, [iv])
                return ()

            lax.fori_loop(0, EPW // 16, body, (), unroll=4)
            pltpu.sync_copy(out_v, dst_hbm.at[pl.ds(base, EPW)])

        one(row_hbm, zr_hbm)
        one(col_hbm, zc_hbm)

    return k(z, row, col)


def _tc_tables(embp, sW1, sb1, sW2, sb2, tW1, tb1, tW2, tb2,
               eW1_top, eW1_mid, eW1_bot, dW2, db2, eb1):
    def k(embp_r, sW1_r, sb1_r, sW2_r, sb2_r, tW1_r, tb1_r, tW2_r, tb2_r,
          eW1t_r, eW1m_r, eW1b_r, dW2_r, db2_r, eb1_r,
          sc_o, tc_o, m_o, c_o):
        f32 = jnp.float32
        emb_v = embp_r[...]
        s_all = _silu(jnp.dot(emb_v, sW1_r[...], preferred_element_type=f32) + sb1_r[...])
        s_all = jnp.dot(s_all, sW2_r[...], preferred_element_type=f32) + sb2_r[...]
        sc_o[...] = jnp.dot(s_all, eW1m_r[...], preferred_element_type=f32)
        t_all = _silu(jnp.dot(emb_v, tW1_r[...], preferred_element_type=f32) + tb1_r[...])
        t_all = jnp.dot(t_all, tW2_r[...], preferred_element_type=f32) + tb2_r[...]
        tc_o[...] = jnp.dot(t_all, eW1b_r[...], preferred_element_type=f32)
        m_o[...] = jnp.dot(dW2_r[...], eW1t_r[...], preferred_element_type=f32)
        c_o[...] = eb1_r[...] + jnp.dot(db2_r[...], eW1t_r[...], preferred_element_type=f32)

    return pl.pallas_call(
        k,
        out_shape=[jax.ShapeDtypeStruct((128, 128), jnp.float32),
                   jax.ShapeDtypeStruct((128, 128), jnp.float32),
                   jax.ShapeDtypeStruct((128, 128), jnp.float32),
                   jax.ShapeDtypeStruct((1, 128), jnp.float32)],
    )(embp, sW1, sb1.reshape(1, 128), sW2, sb2.reshape(1, 128),
      tW1, tb1.reshape(1, 128), tW2, tb2.reshape(1, 128),
      eW1_top, eW1_mid, eW1_bot, dW2, db2.reshape(1, 128), eb1.reshape(1, 128))


def _tc_edges(w3, zr3, zc3, dW1p, db1, s_con, t_con, M, cvec, eW2, eb2):
    def k(w_r, zr_r, zc_r, dW1_r, db1_r, sc_r, tc_r, m_r, c_r, eW2_r, eb2_r, out_r):
        f32 = jnp.float32
        r = w_r[0]
        n = lax.broadcasted_iota(jnp.int32, (32, EB), 0).astype(f32) + 1.0
        sins = jnp.sin(n * ((jnp.pi / CUTOFF) * r))
        basis_t = (jnp.sqrt(2.0 / CUTOFF) / r) * sins
        hpre = lax.dot_general(basis_t, dW1_r[...],
                               (((0,), (0,)), ((), ())),
                               preferred_element_type=f32)
        h = _silu(hpre + db1_r[...])
        pre = jnp.dot(h, m_r[...], preferred_element_type=f32)
        lanes = lax.broadcasted_iota(jnp.int32, (EB, 128), 1)
        ohr = (lanes == zr_r[0, 0, :].reshape(EB, 1)).astype(f32)
        ohc = (lanes == zc_r[0, 0, :].reshape(EB, 1)).astype(f32)
        pre = pre + jnp.dot(ohr, sc_r[...], preferred_element_type=f32)
        pre = pre + jnp.dot(ohc, tc_r[...], preferred_element_type=f32)
        pre = _silu(pre + c_r[...])
        out_r[...] = jnp.dot(pre, eW2_r[...], preferred_element_type=f32) + eb2_r[...]

    full = lambda s: pl.BlockSpec(s, lambda i: (0,) * len(s))
    return pl.pallas_call(
        k,
        grid=(NBLK,),
        in_specs=[
            pl.BlockSpec((1, 1, EB), lambda i: (i, 0, 0)),
            pl.BlockSpec((1, 1, EB), lambda i: (i, 0, 0)),
            pl.BlockSpec((1, 1, EB), lambda i: (i, 0, 0)),
            full((32, 128)), full((1, 128)), full((128, 128)),
            full((128, 128)), full((128, 128)), full((1, 128)),
            full((128, 128)), full((1, 128)),
        ],
        out_specs=pl.BlockSpec((EB, 128), lambda i: (i, 0)),
        out_shape=jax.ShapeDtypeStruct((E, 128), jnp.float32),
    )(w3, zr3, zc3, dW1p, db1, s_con, t_con, M, cvec, eW2, eb2)


def _sc_scatter(ef4, row3, zeros):
    mesh = plsc.VectorSubcoreMesh(core_axis_name="c", subcore_axis_name="s")

    @functools.partial(
        pl.kernel,
        out_type=jax.ShapeDtypeStruct((NC, N, 128), jnp.float32),
        mesh=mesh,
        scratch_types=[pltpu.VMEM((NCHUNK, CHUNK), jnp.int32),
                       pltpu.VMEM((CHUNK, 128), jnp.float32),
                       pltpu.VMEM_SHARED((N, 128), jnp.float32)],
    )
    def k(ef_hbm, row_hbm, zero_hbm, out_hbm, idx_v, buf_v, acc_sh):
        c = lax.axis_index("c")
        s = lax.axis_index("s")
        wid = s * NC + c

        @pl.when(s < 10)
        def _zero():
            pltpu.sync_copy(zero_hbm, acc_sh.at[pl.ds(s * 1000, 1000)])

        plsc.subcore_barrier()
        pltpu.sync_copy(row_hbm.at[wid], idx_v)

        def body(j, _):
            pltpu.sync_copy(ef_hbm.at[wid, j], buf_v)
            pltpu.sync_copy(buf_v, acc_sh.at[idx_v.at[j]], add=True)
            return ()

        lax.fori_loop(0, NCHUNK, body, ())
        plsc.subcore_barrier()

        @pl.when(s < 10)
        def _dump():
            pltpu.sync_copy(acc_sh.at[pl.ds(s * 1000, 1000)],
                            out_hbm.at[c, pl.ds(s * 1000, 1000)])

    return k(ef4, row3, zeros)


def _tc_combine(partials):
    def k(p_r, out_r):
        out_r[...] = p_r[0] + p_r[1]

    return pl.pallas_call(
        k,
        grid=(10,),
        in_specs=[pl.BlockSpec((2, N // 10, 128), lambda i: (0, i, 0))],
        out_specs=pl.BlockSpec((N // 10, 128), lambda i: (i, 0)),
        out_shape=jax.ShapeDtypeStruct((N, 128), jnp.float32),
    )(partials)


def kernel(z, edge_index, edge_weight, emb, dW1, db1, dW2, db2, sW1, sb1,
           sW2, sb2, tW1, tb1, tW2, tb2, eW1, eb1, eW2, eb2):
    z = z.astype(jnp.int32)
    row = edge_index[0].astype(jnp.int32)
    col = edge_index[1].astype(jnp.int32)

    zr, zc = _sc_gather_z(z, row, col)

    embp = jnp.zeros((128, 128), jnp.float32).at[:120].set(emb)
    s_con, t_con, M, cvec = _tc_tables(
        embp, sW1, sb1, sW2, sb2, tW1, tb1, tW2, tb2,
        eW1[0:128], eW1[128:256], eW1[256:384], dW2, db2, eb1)

    dW1p = jnp.zeros((32, 128), jnp.float32).at[:NB].set(dW1)
    ef = _tc_edges(edge_weight.reshape(NBLK, 1, EB),
                   zr.reshape(NBLK, 1, EB), zc.reshape(NBLK, 1, EB),
                   dW1p, db1.reshape(1, 128), s_con, t_con, M, cvec,
                   eW2, eb2.reshape(1, 128))

    partials = _sc_scatter(ef.reshape(NW, NCHUNK, CHUNK, 128),
                           row.reshape(NW, NCHUNK, CHUNK),
                           jnp.zeros((1000, 128), jnp.float32))
    return _tc_combine(partials)


# speedup vs baseline: 5.4395x; 1.1797x over previous
import functools

import jax
import jax.numpy as jnp
from jax import lax
from jax.experimental import pallas as pl
from jax.experimental.pallas import tpu as pltpu
from jax.experimental.pallas import tpu_sc as plsc

N = 10000
E = 320000
NB = 20
CUTOFF = 5.0

NC = 2
NS = 16
NW = NC * NS
EPW = E // NW
CHUNK = 125
NCHUNK = EPW // CHUNK
ROWS_PER_TILE = N // NS

EB = 2560
NBLK = E // EB


def _silu(x):
    return x * jax.nn.sigmoid(x)


def _sc_gather_z(z, row, col):
    mesh = plsc.VectorSubcoreMesh(core_axis_name="c", subcore_axis_name="s")

    @functools.partial(
        pl.kernel,
        out_type=[jax.ShapeDtypeStruct((E,), jnp.int32),
                  jax.ShapeDtypeStruct((E,), jnp.int32)],
        mesh=mesh,
        scratch_types=[pltpu.VMEM((N,), jnp.int32),
                       pltpu.VMEM((EPW,), jnp.int32),
                       pltpu.VMEM((EPW,), jnp.int32)],
        compiler_params=pltpu.CompilerParams(needs_layout_passes=False),
    )
    def k(z_hbm, row_hbm, col_hbm, zr_hbm, zc_hbm, z_v, idx_v, out_v):
        wid = lax.axis_index("s") * NC + lax.axis_index("c")
        base = wid * EPW
        pltpu.sync_copy(z_hbm, z_v)

        def one(src_hbm, dst_hbm):
            pltpu.sync_copy(src_hbm.at[pl.ds(base, EPW)], idx_v)

            def body(i, _):
                iv = idx_v[pl.ds(i * 16, 16)]
                out_v[pl.ds(i * 16, 16)] = plsc.load_gather(z_v, [iv])
                return ()

            lax.fori_loop(0, EPW // 16, body, (), unroll=4)
            pltpu.sync_copy(out_v, dst_hbm.at[pl.ds(base, EPW)])

        one(row_hbm, zr_hbm)
        one(col_hbm, zc_hbm)

    return k(z, row, col)


def _tc_tables(embp, sW1, sb1, sW2, sb2, tW1, tb1, tW2, tb2,
               eW1_top, eW1_mid, eW1_bot, dW2, db2, eb1):
    def k(embp_r, sW1_r, sb1_r, sW2_r, sb2_r, tW1_r, tb1_r, tW2_r, tb2_r,
          eW1t_r, eW1m_r, eW1b_r, dW2_r, db2_r, eb1_r,
          sc_o, tc_o, m_o, c_o):
        f32 = jnp.float32
        emb_v = embp_r[...]
        s_all = _silu(jnp.dot(emb_v, sW1_r[...], preferred_element_type=f32) + sb1_r[...])
        s_all = jnp.dot(s_all, sW2_r[...], preferred_element_type=f32) + sb2_r[...]
        sc_o[...] = jnp.dot(s_all, eW1m_r[...], preferred_element_type=f32)
        t_all = _silu(jnp.dot(emb_v, tW1_r[...], preferred_element_type=f32) + tb1_r[...])
        t_all = jnp.dot(t_all, tW2_r[...], preferred_element_type=f32) + tb2_r[...]
        tc_o[...] = jnp.dot(t_all, eW1b_r[...], preferred_element_type=f32)
        m_o[...] = jnp.dot(dW2_r[...], eW1t_r[...], preferred_element_type=f32)
        c_o[...] = eb1_r[...] + jnp.dot(db2_r[...], eW1t_r[...], preferred_element_type=f32)

    return pl.pallas_call(
        k,
        out_shape=[jax.ShapeDtypeStruct((128, 128), jnp.float32),
                   jax.ShapeDtypeStruct((128, 128), jnp.float32),
                   jax.ShapeDtypeStruct((128, 128), jnp.float32),
                   jax.ShapeDtypeStruct((1, 128), jnp.float32)],
    )(embp, sW1, sb1.reshape(1, 128), sW2, sb2.reshape(1, 128),
      tW1, tb1.reshape(1, 128), tW2, tb2.reshape(1, 128),
      eW1_top, eW1_mid, eW1_bot, dW2, db2.reshape(1, 128), eb1.reshape(1, 128))


def _tc_edges(w3, zr3, zc3, dW1p, db1, s_con, t_con, M, cvec, eW2, eb2):
    def k(w_r, zr_r, zc_r, dW1_r, db1_r, sc_r, tc_r, m_r, c_r, eW2_r, eb2_r, out_r):
        f32 = jnp.float32
        r = w_r[0]
        n = lax.broadcasted_iota(jnp.int32, (32, EB), 0).astype(f32) + 1.0
        x = n * ((jnp.pi / CUTOFF) * r)
        k = jnp.round(x * 0.15915493667125702)
        xr = (x - k * 6.2831854820251465) - k * (-1.7484555314695172e-07)
        x2 = xr * xr
        p = jnp.float32(-2.069779872493349e-08)
        p = p * x2 + jnp.float32(2.708822857390436e-06)
        p = p * x2 + jnp.float32(-0.0001981762360091944)
        p = p * x2 + jnp.float32(0.008332791218600519)
        p = p * x2 + jnp.float32(-0.16666621064339257)
        p = p * x2 + jnp.float32(0.9999999376350313)
        sins = xr * p
        basis_t = (jnp.sqrt(2.0 / CUTOFF) / r) * sins
        hpre = lax.dot_general(basis_t, dW1_r[...],
                               (((0,), (0,)), ((), ())),
                               preferred_element_type=f32)
        h = _silu(hpre + db1_r[...])
        pre = jnp.dot(h, m_r[...], preferred_element_type=f32)
        lanes = lax.broadcasted_iota(jnp.int32, (EB, 128), 1)
        ohr = (lanes == zr_r[0, 0, :].reshape(EB, 1)).astype(f32)
        ohc = (lanes == zc_r[0, 0, :].reshape(EB, 1)).astype(f32)
        pre = pre + jnp.dot(ohr, sc_r[...], preferred_element_type=f32)
        pre = pre + jnp.dot(ohc, tc_r[...], preferred_element_type=f32)
        pre = _silu(pre + c_r[...])
        out_r[...] = jnp.dot(pre, eW2_r[...], preferred_element_type=f32) + eb2_r[...]

    full = lambda s: pl.BlockSpec(s, lambda i: (0,) * len(s))
    return pl.pallas_call(
        k,
        grid=(NBLK,),
        in_specs=[
            pl.BlockSpec((1, 1, EB), lambda i: (i, 0, 0)),
            pl.BlockSpec((1, 1, EB), lambda i: (i, 0, 0)),
            pl.BlockSpec((1, 1, EB), lambda i: (i, 0, 0)),
            full((32, 128)), full((1, 128)), full((128, 128)),
            full((128, 128)), full((128, 128)), full((1, 128)),
            full((128, 128)), full((1, 128)),
        ],
        out_specs=pl.BlockSpec((EB, 128), lambda i: (i, 0)),
        out_shape=jax.ShapeDtypeStruct((E, 128), jnp.float32),
    )(w3, zr3, zc3, dW1p, db1, s_con, t_con, M, cvec, eW2, eb2)


def _sc_scatter(ef4, row3, zeros):
    mesh = plsc.VectorSubcoreMesh(core_axis_name="c", subcore_axis_name="s")

    @functools.partial(
        pl.kernel,
        out_type=jax.ShapeDtypeStruct((NC, N, 128), jnp.float32),
        mesh=mesh,
        scratch_types=[pltpu.VMEM((NCHUNK, CHUNK), jnp.int32),
                       pltpu.VMEM((CHUNK, 128), jnp.float32),
                       pltpu.VMEM_SHARED((N, 128), jnp.float32)],
    )
    def k(ef_hbm, row_hbm, zero_hbm, out_hbm, idx_v, buf_v, acc_sh):
        c = lax.axis_index("c")
        s = lax.axis_index("s")
        wid = s * NC + c

        @pl.when(s < 10)
        def _zero():
            pltpu.sync_copy(zero_hbm, acc_sh.at[pl.ds(s * 1000, 1000)])

        plsc.subcore_barrier()
        pltpu.sync_copy(row_hbm.at[wid], idx_v)

        def body(j, _):
            pltpu.sync_copy(ef_hbm.at[wid, j], buf_v)
            pltpu.sync_copy(buf_v, acc_sh.at[idx_v.at[j]], add=True)
            return ()

        lax.fori_loop(0, NCHUNK, body, ())
        plsc.subcore_barrier()

        @pl.when(s < 10)
        def _dump():
            pltpu.sync_copy(acc_sh.at[pl.ds(s * 1000, 1000)],
                            out_hbm.at[c, pl.ds(s * 1000, 1000)])

    return k(ef4, row3, zeros)


def _tc_combine(partials):
    def k(p_r, out_r):
        out_r[...] = p_r[0] + p_r[1]

    return pl.pallas_call(
        k,
        grid=(10,),
        in_specs=[pl.BlockSpec((2, N // 10, 128), lambda i: (0, i, 0))],
        out_specs=pl.BlockSpec((N // 10, 128), lambda i: (i, 0)),
        out_shape=jax.ShapeDtypeStruct((N, 128), jnp.float32),
    )(partials)


def kernel(z, edge_index, edge_weight, emb, dW1, db1, dW2, db2, sW1, sb1,
           sW2, sb2, tW1, tb1, tW2, tb2, eW1, eb1, eW2, eb2):
    z = z.astype(jnp.int32)
    row = edge_index[0].astype(jnp.int32)
    col = edge_index[1].astype(jnp.int32)

    zr, zc = _sc_gather_z(z, row, col)

    embp = jnp.zeros((128, 128), jnp.float32).at[:120].set(emb)
    s_con, t_con, M, cvec = _tc_tables(
        embp, sW1, sb1, sW2, sb2, tW1, tb1, tW2, tb2,
        eW1[0:128], eW1[128:256], eW1[256:384], dW2, db2, eb1)

    dW1p = jnp.zeros((32, 128), jnp.float32).at[:NB].set(dW1)
    ef = _tc_edges(edge_weight.reshape(NBLK, 1, EB),
                   zr.reshape(NBLK, 1, EB), zc.reshape(NBLK, 1, EB),
                   dW1p, db1.reshape(1, 128), s_con, t_con, M, cvec,
                   eW2, eb2.reshape(1, 128))

    partials = _sc_scatter(ef.reshape(NW, NCHUNK, CHUNK, 128),
                           row.reshape(NW, NCHUNK, CHUNK),
                           jnp.zeros((1000, 128), jnp.float32))
    return _tc_combine(partials)


# speedup vs baseline: 8.1248x; 1.4937x over previous
import functools

import jax
import jax.numpy as jnp
from jax import lax
from jax.experimental import pallas as pl
from jax.experimental.pallas import tpu as pltpu
from jax.experimental.pallas import tpu_sc as plsc

N = 10000
E = 320000
NB = 20
CUTOFF = 5.0

NC = 2
NS = 16
NW = NC * NS
EPW = E // NW
CHUNK = 80
NCHUNK = EPW // CHUNK
ROWS_PER_TILE = N // NS

EB = 2560
NBLK = E // EB


def _silu(x):
    return x * jax.nn.sigmoid(x)


def _sc_gather_z(z, row, col):
    mesh = plsc.VectorSubcoreMesh(core_axis_name="c", subcore_axis_name="s")

    @functools.partial(
        pl.kernel,
        out_type=[jax.ShapeDtypeStruct((E,), jnp.int32),
                  jax.ShapeDtypeStruct((E,), jnp.int32)],
        mesh=mesh,
        scratch_types=[pltpu.VMEM((N,), jnp.int32),
                       pltpu.VMEM((EPW,), jnp.int32),
                       pltpu.VMEM((EPW,), jnp.int32)],
        compiler_params=pltpu.CompilerParams(needs_layout_passes=False),
    )
    def k(z_hbm, row_hbm, col_hbm, zr_hbm, zc_hbm, z_v, idx_v, out_v):
        wid = lax.axis_index("s") * NC + lax.axis_index("c")
        base = wid * EPW
        pltpu.sync_copy(z_hbm, z_v)

        def one(src_hbm, dst_hbm):
            pltpu.sync_copy(src_hbm.at[pl.ds(base, EPW)], idx_v)

            def body(i, _):
                iv = idx_v[pl.ds(i * 16, 16)]
                out_v[pl.ds(i * 16, 16)] = plsc.load_gather(z_v, [iv])
                return ()

            lax.fori_loop(0, EPW // 16, body, (), unroll=4)
            pltpu.sync_copy(out_v, dst_hbm.at[pl.ds(base, EPW)])

        one(row_hbm, zr_hbm)
        one(col_hbm, zc_hbm)

    return k(z, row, col)


def _tc_tables(embp, sW1, sb1, sW2, sb2, tW1, tb1, tW2, tb2,
               eW1_top, eW1_mid, eW1_bot, dW2, db2, eb1):
    def k(embp_r, sW1_r, sb1_r, sW2_r, sb2_r, tW1_r, tb1_r, tW2_r, tb2_r,
          eW1t_r, eW1m_r, eW1b_r, dW2_r, db2_r, eb1_r,
          sc_o, tc_o, m_o, c_o):
        f32 = jnp.float32
        emb_v = embp_r[...]
        s_all = _silu(jnp.dot(emb_v, sW1_r[...], preferred_element_type=f32) + sb1_r[...])
        s_all = jnp.dot(s_all, sW2_r[...], preferred_element_type=f32) + sb2_r[...]
        sc_o[...] = jnp.dot(s_all, eW1m_r[...], preferred_element_type=f32)
        t_all = _silu(jnp.dot(emb_v, tW1_r[...], preferred_element_type=f32) + tb1_r[...])
        t_all = jnp.dot(t_all, tW2_r[...], preferred_element_type=f32) + tb2_r[...]
        tc_o[...] = jnp.dot(t_all, eW1b_r[...], preferred_element_type=f32)
        m_o[...] = jnp.dot(dW2_r[...], eW1t_r[...], preferred_element_type=f32)
        c_o[...] = eb1_r[...] + jnp.dot(db2_r[...], eW1t_r[...], preferred_element_type=f32)

    return pl.pallas_call(
        k,
        out_shape=[jax.ShapeDtypeStruct((128, 128), jnp.float32),
                   jax.ShapeDtypeStruct((128, 128), jnp.float32),
                   jax.ShapeDtypeStruct((128, 128), jnp.float32),
                   jax.ShapeDtypeStruct((1, 128), jnp.float32)],
    )(embp, sW1, sb1.reshape(1, 128), sW2, sb2.reshape(1, 128),
      tW1, tb1.reshape(1, 128), tW2, tb2.reshape(1, 128),
      eW1_top, eW1_mid, eW1_bot, dW2, db2.reshape(1, 128), eb1.reshape(1, 128))


def _tc_edges(w3, zr3, zc3, dW1p, db1, s_con, t_con, M, cvec, eW2, eb2):
    def k(w_r, zr_r, zc_r, dW1_r, db1_r, sc_r, tc_r, m_r, c_r, eW2_r, eb2_r, out_r):
        f32 = jnp.float32
        r = w_r[0]
        n = lax.broadcasted_iota(jnp.int32, (32, EB), 0).astype(f32) + 1.0
        x = n * ((jnp.pi / CUTOFF) * r)
        k = jnp.round(x * 0.15915493667125702)
        xr = (x - k * 6.2831854820251465) - k * (-1.7484555314695172e-07)
        x2 = xr * xr
        p = jnp.float32(-2.069779872493349e-08)
        p = p * x2 + jnp.float32(2.708822857390436e-06)
        p = p * x2 + jnp.float32(-0.0001981762360091944)
        p = p * x2 + jnp.float32(0.008332791218600519)
        p = p * x2 + jnp.float32(-0.16666621064339257)
        p = p * x2 + jnp.float32(0.9999999376350313)
        sins = xr * p
        basis_t = (jnp.sqrt(2.0 / CUTOFF) / r) * sins
        hpre = lax.dot_general(basis_t, dW1_r[...],
                               (((0,), (0,)), ((), ())),
                               preferred_element_type=f32)
        h = _silu(hpre + db1_r[...])
        pre = jnp.dot(h, m_r[...], preferred_element_type=f32)
        lanes = lax.broadcasted_iota(jnp.int32, (EB, 128), 1)
        ohr = (lanes == zr_r[0, 0, :].reshape(EB, 1)).astype(f32)
        ohc = (lanes == zc_r[0, 0, :].reshape(EB, 1)).astype(f32)
        pre = pre + jnp.dot(ohr, sc_r[...], preferred_element_type=f32)
        pre = pre + jnp.dot(ohc, tc_r[...], preferred_element_type=f32)
        pre = _silu(pre + c_r[...])
        out_r[...] = jnp.dot(pre, eW2_r[...], preferred_element_type=f32) + eb2_r[...]

    full = lambda s: pl.BlockSpec(s, lambda i: (0,) * len(s))
    return pl.pallas_call(
        k,
        grid=(NBLK,),
        in_specs=[
            pl.BlockSpec((1, 1, EB), lambda i: (i, 0, 0)),
            pl.BlockSpec((1, 1, EB), lambda i: (i, 0, 0)),
            pl.BlockSpec((1, 1, EB), lambda i: (i, 0, 0)),
            full((32, 128)), full((1, 128)), full((128, 128)),
            full((128, 128)), full((128, 128)), full((1, 128)),
            full((128, 128)), full((1, 128)),
        ],
        out_specs=pl.BlockSpec((EB, 128), lambda i: (i, 0)),
        out_shape=jax.ShapeDtypeStruct((E, 128), jnp.float32),
    )(w3, zr3, zc3, dW1p, db1, s_con, t_con, M, cvec, eW2, eb2)


def _sc_scatter(ef, row3, zeros):
    mesh = plsc.VectorSubcoreMesh(core_axis_name="c", subcore_axis_name="s")

    @functools.partial(
        pl.kernel,
        out_type=jax.ShapeDtypeStruct((NC, N, 128), jnp.float32),
        mesh=mesh,
        scratch_types=[pltpu.VMEM((NCHUNK, CHUNK), jnp.int32),
                       pltpu.VMEM((CHUNK, 128), jnp.float32),
                       pltpu.VMEM((CHUNK, 128), jnp.float32),
                       pltpu.VMEM_SHARED((N, 128), jnp.float32),
                       pltpu.SemaphoreType.DMA,
                       pltpu.SemaphoreType.DMA],
    )
    def k(ef_hbm, row_hbm, zero_hbm, out_hbm, idx_v, buf0, buf1, acc_sh,
          sem0, sem1):
        c = lax.axis_index("c")
        s = lax.axis_index("s")
        wid = s * NC + c
        base = wid * EPW

        @pl.when(s < 10)
        def _zero():
            pltpu.sync_copy(zero_hbm, acc_sh.at[pl.ds(s * 1000, 1000)])

        pltpu.sync_copy(row_hbm.at[wid], idx_v)
        plsc.subcore_barrier()

        def fetch(j, buf, sem):
            off = pl.multiple_of(base + j * CHUNK, 8)
            pltpu.async_copy(ef_hbm.at[pl.ds(off, CHUNK)], buf, sem)

        def drain(buf, sem):
            pltpu.make_async_copy(ef_hbm.at[pl.ds(base, CHUNK)], buf, sem).wait()

        def scat(j, buf):
            pltpu.sync_copy(buf, acc_sh.at[idx_v.at[j]], add=True)

        fetch(0, buf0, sem0)

        def body(jj, _):
            j = 2 * jj
            fetch(j + 1, buf1, sem1)
            drain(buf0, sem0)
            scat(j, buf0)
            fetch(j + 2, buf0, sem0)
            drain(buf1, sem1)
            scat(j + 1, buf1)
            return ()

        lax.fori_loop(0, (NCHUNK - 1) // 2, body, ())
        drain(buf0, sem0)
        scat(NCHUNK - 1, buf0)
        plsc.subcore_barrier()

        @pl.when(s < 10)
        def _dump():
            pltpu.sync_copy(acc_sh.at[pl.ds(s * 1000, 1000)],
                            out_hbm.at[c, pl.ds(s * 1000, 1000)])

    return k(ef, row3, zeros)


def _tc_combine(partials):
    def k(p_r, out_r):
        out_r[...] = p_r[0] + p_r[1]

    return pl.pallas_call(
        k,
        grid=(10,),
        in_specs=[pl.BlockSpec((2, N // 10, 128), lambda i: (0, i, 0))],
        out_specs=pl.BlockSpec((N // 10, 128), lambda i: (i, 0)),
        out_shape=jax.ShapeDtypeStruct((N, 128), jnp.float32),
    )(partials)


def kernel(z, edge_index, edge_weight, emb, dW1, db1, dW2, db2, sW1, sb1,
           sW2, sb2, tW1, tb1, tW2, tb2, eW1, eb1, eW2, eb2):
    z = z.astype(jnp.int32)
    row = edge_index[0].astype(jnp.int32)
    col = edge_index[1].astype(jnp.int32)

    zr, zc = _sc_gather_z(z, row, col)

    embp = jnp.zeros((128, 128), jnp.float32).at[:120].set(emb)
    s_con, t_con, M, cvec = _tc_tables(
        embp, sW1, sb1, sW2, sb2, tW1, tb1, tW2, tb2,
        eW1[0:128], eW1[128:256], eW1[256:384], dW2, db2, eb1)

    dW1p = jnp.zeros((32, 128), jnp.float32).at[:NB].set(dW1)
    ef = _tc_edges(edge_weight.reshape(NBLK, 1, EB),
                   zr.reshape(NBLK, 1, EB), zc.reshape(NBLK, 1, EB),
                   dW1p, db1.reshape(1, 128), s_con, t_con, M, cvec,
                   eW2, eb2.reshape(1, 128))

    partials = _sc_scatter(ef, row.reshape(NW, NCHUNK, CHUNK),
                           jnp.zeros((1000, 128), jnp.float32))
    return _tc_combine(partials)


# speedup vs baseline: 8.6584x; 1.0657x over previous
import functools

import jax
import jax.numpy as jnp
from jax import lax
from jax.experimental import pallas as pl
from jax.experimental.pallas import tpu as pltpu
from jax.experimental.pallas import tpu_sc as plsc

N = 10000
E = 320000
NB = 20
CUTOFF = 5.0

NC = 2
NS = 16
NW = NC * NS
EPW = E // NW
CHUNK = 80
NCHUNK = EPW // CHUNK
ROWS_PER_TILE = N // NS

EB = 2560
NBLK = E // EB


def _silu(x):
    return x * jax.nn.sigmoid(x)


def _sc_gather_z(z, row, col):
    mesh = plsc.VectorSubcoreMesh(core_axis_name="c", subcore_axis_name="s")

    @functools.partial(
        pl.kernel,
        out_type=[jax.ShapeDtypeStruct((E,), jnp.int32),
                  jax.ShapeDtypeStruct((E,), jnp.int32)],
        mesh=mesh,
        scratch_types=[pltpu.VMEM((N,), jnp.int32),
                       pltpu.VMEM((EPW,), jnp.int32),
                       pltpu.VMEM((EPW,), jnp.int32)],
        compiler_params=pltpu.CompilerParams(needs_layout_passes=False),
    )
    def k(z_hbm, row_hbm, col_hbm, zr_hbm, zc_hbm, z_v, idx_v, out_v):
        wid = lax.axis_index("s") * NC + lax.axis_index("c")
        base = wid * EPW
        pltpu.sync_copy(z_hbm, z_v)

        def one(src_hbm, dst_hbm):
            pltpu.sync_copy(src_hbm.at[pl.ds(base, EPW)], idx_v)

            def body(i, _):
                iv = idx_v[pl.ds(i * 16, 16)]
                out_v[pl.ds(i * 16, 16)] = plsc.load_gather(z_v, [iv])
                return ()

            lax.fori_loop(0, EPW // 16, body, (), unroll=4)
            pltpu.sync_copy(out_v, dst_hbm.at[pl.ds(base, EPW)])

        one(row_hbm, zr_hbm)
        one(col_hbm, zc_hbm)

    return k(z, row, col)


def _tc_tables(embp, sW1, sb1, sW2, sb2, tW1, tb1, tW2, tb2,
               eW1_top, eW1_mid, eW1_bot, dW2, db2, eb1):
    def k(embp_r, sW1_r, sb1_r, sW2_r, sb2_r, tW1_r, tb1_r, tW2_r, tb2_r,
          eW1t_r, eW1m_r, eW1b_r, dW2_r, db2_r, eb1_r,
          sc_o, tc_o, m_o, c_o):
        f32 = jnp.float32
        emb_v = embp_r[...]
        s_all = _silu(jnp.dot(emb_v, sW1_r[...], preferred_element_type=f32) + sb1_r[...])
        s_all = jnp.dot(s_all, sW2_r[...], preferred_element_type=f32) + sb2_r[...]
        sc_o[...] = jnp.dot(s_all, eW1m_r[...], preferred_element_type=f32).astype(jnp.bfloat16)
        t_all = _silu(jnp.dot(emb_v, tW1_r[...], preferred_element_type=f32) + tb1_r[...])
        t_all = jnp.dot(t_all, tW2_r[...], preferred_element_type=f32) + tb2_r[...]
        tc_o[...] = jnp.dot(t_all, eW1b_r[...], preferred_element_type=f32).astype(jnp.bfloat16)
        m_o[...] = jnp.dot(dW2_r[...], eW1t_r[...], preferred_element_type=f32)
        c_o[...] = eb1_r[...] + jnp.dot(db2_r[...], eW1t_r[...], preferred_element_type=f32)

    return pl.pallas_call(
        k,
        out_shape=[jax.ShapeDtypeStruct((128, 128), jnp.bfloat16),
                   jax.ShapeDtypeStruct((128, 128), jnp.bfloat16),
                   jax.ShapeDtypeStruct((128, 128), jnp.float32),
                   jax.ShapeDtypeStruct((1, 128), jnp.float32)],
    )(embp, sW1, sb1.reshape(1, 128), sW2, sb2.reshape(1, 128),
      tW1, tb1.reshape(1, 128), tW2, tb2.reshape(1, 128),
      eW1_top, eW1_mid, eW1_bot, dW2, db2.reshape(1, 128), eb1.reshape(1, 128))


def _tc_edges(w3, zr3, zc3, dW1p, db1, s_con, t_con, M, cvec, eW2, eb2):
    def k(w_r, zr_r, zc_r, dW1_r, db1_r, sc_r, tc_r, m_r, c_r, eW2_r, eb2_r, out_r):
        f32 = jnp.float32
        r = w_r[0]
        n = lax.broadcasted_iota(jnp.int32, (32, EB), 0).astype(f32) + 1.0
        x = n * ((jnp.pi / CUTOFF) * r)
        k = jnp.round(x * 0.15915493667125702)
        xr = (x - k * 6.2831854820251465) - k * (-1.7484555314695172e-07)
        x2 = xr * xr
        p = jnp.float32(-2.069779872493349e-08)
        p = p * x2 + jnp.float32(2.708822857390436e-06)
        p = p * x2 + jnp.float32(-0.0001981762360091944)
        p = p * x2 + jnp.float32(0.008332791218600519)
        p = p * x2 + jnp.float32(-0.16666621064339257)
        p = p * x2 + jnp.float32(0.9999999376350313)
        sins = xr * p
        basis_t = (jnp.sqrt(2.0 / CUTOFF) / r) * sins
        hpre = lax.dot_general(basis_t, dW1_r[...],
                               (((0,), (0,)), ((), ())),
                               preferred_element_type=f32)
        h = _silu(hpre + db1_r[...])
        pre = jnp.dot(h, m_r[...], preferred_element_type=f32)
        lanes = lax.broadcasted_iota(jnp.int32, (EB, 128), 1)
        ohr = (lanes == zr_r[0, 0, :].reshape(EB, 1)).astype(jnp.bfloat16)
        ohc = (lanes == zc_r[0, 0, :].reshape(EB, 1)).astype(jnp.bfloat16)
        pre = pre + jnp.dot(ohr, sc_r[...], preferred_element_type=f32)
        pre = pre + jnp.dot(ohc, tc_r[...], preferred_element_type=f32)
        pre = _silu(pre + c_r[...])
        out_r[...] = jnp.dot(pre, eW2_r[...], preferred_element_type=f32) + eb2_r[...]

    nblk = w3.shape[0]
    full = lambda s: pl.BlockSpec(s, lambda i: (0,) * len(s))
    return pl.pallas_call(
        k,
        grid=(nblk,),
        in_specs=[
            pl.BlockSpec((1, 1, EB), lambda i: (i, 0, 0)),
            pl.BlockSpec((1, 1, EB), lambda i: (i, 0, 0)),
            pl.BlockSpec((1, 1, EB), lambda i: (i, 0, 0)),
            full((32, 128)), full((1, 128)), full((128, 128)),
            full((128, 128)), full((128, 128)), full((1, 128)),
            full((128, 128)), full((1, 128)),
        ],
        out_specs=pl.BlockSpec((EB, 128), lambda i: (i, 0)),
        out_shape=jax.ShapeDtypeStruct((nblk * EB, 128), jnp.float32),
    )(w3, zr3, zc3, dW1p, db1, s_con, t_con, M, cvec, eW2, eb2)


def _sc_scatter(ef, row3, zeros):
    mesh = plsc.VectorSubcoreMesh(core_axis_name="c", subcore_axis_name="s")
    nchunk = row3.shape[1]
    epw = nchunk * CHUNK

    @functools.partial(
        pl.kernel,
        out_type=jax.ShapeDtypeStruct((NC, N, 128), jnp.float32),
        mesh=mesh,
        scratch_types=[pltpu.VMEM((nchunk, CHUNK), jnp.int32),
                       pltpu.VMEM((CHUNK, 128), jnp.float32),
                       pltpu.VMEM((CHUNK, 128), jnp.float32),
                       pltpu.VMEM_SHARED((N, 128), jnp.float32),
                       pltpu.SemaphoreType.DMA,
                       pltpu.SemaphoreType.DMA],
    )
    def k(ef_hbm, row_hbm, zero_hbm, out_hbm, idx_v, buf0, buf1, acc_sh,
          sem0, sem1):
        c = lax.axis_index("c")
        s = lax.axis_index("s")
        wid = s * NC + c
        base = wid * epw

        @pl.when(s < 10)
        def _zero():
            pltpu.sync_copy(zero_hbm, acc_sh.at[pl.ds(s * 1000, 1000)])

        pltpu.sync_copy(row_hbm.at[wid], idx_v)
        plsc.subcore_barrier()

        def fetch(j, buf, sem):
            off = pl.multiple_of(base + j * CHUNK, 8)
            pltpu.async_copy(ef_hbm.at[pl.ds(off, CHUNK)], buf, sem)

        def drain(buf, sem):
            pltpu.make_async_copy(ef_hbm.at[pl.ds(base, CHUNK)], buf, sem).wait()

        def scat(j, buf):
            pltpu.sync_copy(buf, acc_sh.at[idx_v.at[j]], add=True)

        fetch(0, buf0, sem0)

        def body(jj, _):
            j = 2 * jj
            fetch(j + 1, buf1, sem1)
            drain(buf0, sem0)
            scat(j, buf0)

            @pl.when(j + 2 < nchunk)
            def _pre():
                fetch(j + 2, buf0, sem0)

            drain(buf1, sem1)
            scat(j + 1, buf1)
            return ()

        lax.fori_loop(0, nchunk // 2, body, ())
        if nchunk % 2:
            drain(buf0, sem0)
            scat(nchunk - 1, buf0)
        plsc.subcore_barrier()

        @pl.when(s < 10)
        def _dump():
            pltpu.sync_copy(acc_sh.at[pl.ds(s * 1000, 1000)],
                            out_hbm.at[c, pl.ds(s * 1000, 1000)])

    return k(ef, row3, zeros)


def _tc_combine(p0, p1):
    def k(a_r, b_r, out_r):
        out_r[...] = (a_r[0] + a_r[1]) + (b_r[0] + b_r[1])

    spec = pl.BlockSpec((2, N // 10, 128), lambda i: (0, i, 0))
    return pl.pallas_call(
        k,
        grid=(10,),
        in_specs=[spec, spec],
        out_specs=pl.BlockSpec((N // 10, 128), lambda i: (i, 0)),
        out_shape=jax.ShapeDtypeStruct((N, 128), jnp.float32),
    )(p0, p1)


def kernel(z, edge_index, edge_weight, emb, dW1, db1, dW2, db2, sW1, sb1,
           sW2, sb2, tW1, tb1, tW2, tb2, eW1, eb1, eW2, eb2):
    z = z.astype(jnp.int32)
    row = edge_index[0].astype(jnp.int32)
    col = edge_index[1].astype(jnp.int32)

    zr, zc = _sc_gather_z(z, row, col)

    embp = jnp.zeros((128, 128), jnp.float32).at[:120].set(emb)
    s_con, t_con, M, cvec = _tc_tables(
        embp, sW1, sb1, sW2, sb2, tW1, tb1, tW2, tb2,
        eW1[0:128], eW1[128:256], eW1[256:384], dW2, db2, eb1)

    dW1p = jnp.zeros((32, 128), jnp.float32).at[:NB].set(dW1)
    zeros = jnp.zeros((1000, 128), jnp.float32)

    nblk0 = 62
    h0 = nblk0 * EB
    parts = []
    for lo, hi, nblk in ((0, h0, nblk0), (h0, E, NBLK - nblk0)):
        ef = _tc_edges(edge_weight[lo:hi].reshape(nblk, 1, EB),
                       zr[lo:hi].reshape(nblk, 1, EB),
                       zc[lo:hi].reshape(nblk, 1, EB),
                       dW1p, db1.reshape(1, 128), s_con, t_con, M, cvec,
                       eW2, eb2.reshape(1, 128))
        nch = (hi - lo) // NW // CHUNK
        parts.append(_sc_scatter(ef, row[lo:hi].reshape(NW, nch, CHUNK), zeros))

    return _tc_combine(parts[0], parts[1])


# speedup vs baseline: 8.7756x; 1.0135x over previous
import functools

import jax
import jax.numpy as jnp
from jax import lax
from jax.experimental import pallas as pl
from jax.experimental.pallas import tpu as pltpu
from jax.experimental.pallas import tpu_sc as plsc

N = 10000
E = 320000
NB = 20
CUTOFF = 5.0

NC = 2
NS = 16
NW = NC * NS
EPW = E // NW
CHUNK = 80
NCHUNK = EPW // CHUNK
ROWS_PER_TILE = N // NS

EB = 2560
NBLK = E // EB


def _silu(x):
    return x * jax.nn.sigmoid(x)


def _sc_gather_z(z, row, col):
    mesh = plsc.VectorSubcoreMesh(core_axis_name="c", subcore_axis_name="s")

    @functools.partial(
        pl.kernel,
        out_type=[jax.ShapeDtypeStruct((E,), jnp.int32),
                  jax.ShapeDtypeStruct((E,), jnp.int32)],
        mesh=mesh,
        scratch_types=[pltpu.VMEM((N,), jnp.int32),
                       pltpu.VMEM((EPW,), jnp.int32),
                       pltpu.VMEM((EPW,), jnp.int32)],
        compiler_params=pltpu.CompilerParams(needs_layout_passes=False),
    )
    def k(z_hbm, row_hbm, col_hbm, zr_hbm, zc_hbm, z_v, idx_v, out_v):
        wid = lax.axis_index("s") * NC + lax.axis_index("c")
        base = wid * EPW
        pltpu.sync_copy(z_hbm, z_v)

        def one(src_hbm, dst_hbm):
            pltpu.sync_copy(src_hbm.at[pl.ds(base, EPW)], idx_v)

            def body(i, _):
                iv = idx_v[pl.ds(i * 16, 16)]
                out_v[pl.ds(i * 16, 16)] = plsc.load_gather(z_v, [iv])
                return ()

            lax.fori_loop(0, EPW // 16, body, (), unroll=4)
            pltpu.sync_copy(out_v, dst_hbm.at[pl.ds(base, EPW)])

        one(row_hbm, zr_hbm)
        one(col_hbm, zc_hbm)

    return k(z, row, col)


def _tc_tables(embp, sW1, sb1, sW2, sb2, tW1, tb1, tW2, tb2,
               eW1_top, eW1_mid, eW1_bot, dW2, db2, eb1):
    def k(embp_r, sW1_r, sb1_r, sW2_r, sb2_r, tW1_r, tb1_r, tW2_r, tb2_r,
          eW1t_r, eW1m_r, eW1b_r, dW2_r, db2_r, eb1_r,
          sc_o, tc_o, m_o, c_o):
        f32 = jnp.float32
        emb_v = embp_r[...]
        s_all = _silu(jnp.dot(emb_v, sW1_r[...], preferred_element_type=f32) + sb1_r[...])
        s_all = jnp.dot(s_all, sW2_r[...], preferred_element_type=f32) + sb2_r[...]
        sc_o[...] = jnp.dot(s_all, eW1m_r[...], preferred_element_type=f32)
        t_all = _silu(jnp.dot(emb_v, tW1_r[...], preferred_element_type=f32) + tb1_r[...])
        t_all = jnp.dot(t_all, tW2_r[...], preferred_element_type=f32) + tb2_r[...]
        tc_o[...] = jnp.dot(t_all, eW1b_r[...], preferred_element_type=f32)
        m_o[...] = jnp.dot(dW2_r[...], eW1t_r[...], preferred_element_type=f32)
        c_o[...] = eb1_r[...] + jnp.dot(db2_r[...], eW1t_r[...], preferred_element_type=f32)

    return pl.pallas_call(
        k,
        out_shape=[jax.ShapeDtypeStruct((128, 128), jnp.float32),
                   jax.ShapeDtypeStruct((128, 128), jnp.float32),
                   jax.ShapeDtypeStruct((128, 128), jnp.float32),
                   jax.ShapeDtypeStruct((1, 128), jnp.float32)],
    )(embp, sW1, sb1.reshape(1, 128), sW2, sb2.reshape(1, 128),
      tW1, tb1.reshape(1, 128), tW2, tb2.reshape(1, 128),
      eW1_top, eW1_mid, eW1_bot, dW2, db2.reshape(1, 128), eb1.reshape(1, 128))


def _tc_edges(w3, zr3, zc3, dW1p, db1, s_con, t_con, M, cvec, eW2, eb2):
    def k(w_r, zr_r, zc_r, dW1_r, db1_r, sc_r, tc_r, m_r, c_r, eW2_r, eb2_r, out_r):
        f32 = jnp.float32
        r = w_r[0]
        n = lax.broadcasted_iota(jnp.int32, (24, EB), 0).astype(f32) + 1.0
        x = n * ((jnp.pi / CUTOFF) * r)
        k = jnp.round(x * 0.15915493667125702)
        xr = (x - k * 6.2831854820251465) - k * (-1.7484555314695172e-07)
        x2 = xr * xr
        p = jnp.float32(-2.069779872493349e-08)
        p = p * x2 + jnp.float32(2.708822857390436e-06)
        p = p * x2 + jnp.float32(-0.0001981762360091944)
        p = p * x2 + jnp.float32(0.008332791218600519)
        p = p * x2 + jnp.float32(-0.16666621064339257)
        p = p * x2 + jnp.float32(0.9999999376350313)
        sins = xr * p
        basis_t = (jnp.sqrt(2.0 / CUTOFF) / r) * sins
        hpre = lax.dot_general(basis_t, dW1_r[...],
                               (((0,), (0,)), ((), ())),
                               preferred_element_type=f32)
        h = _silu(hpre + db1_r[...])
        pre = jnp.dot(h, m_r[...], preferred_element_type=f32)
        lanes = lax.broadcasted_iota(jnp.int32, (EB, 128), 1)
        ohr = (lanes == zr_r[0, 0, :].reshape(EB, 1)).astype(f32)
        ohc = (lanes == zc_r[0, 0, :].reshape(EB, 1)).astype(f32)
        pre = pre + jnp.dot(ohr, sc_r[...], preferred_element_type=f32)
        pre = pre + jnp.dot(ohc, tc_r[...], preferred_element_type=f32)
        pre = _silu(pre + c_r[...])
        out_r[...] = jnp.dot(pre, eW2_r[...], preferred_element_type=f32) + eb2_r[...]

    nblk = w3.shape[0]
    full = lambda s: pl.BlockSpec(s, lambda i: (0,) * len(s))
    return pl.pallas_call(
        k,
        grid=(nblk,),
        in_specs=[
            pl.BlockSpec((1, 1, EB), lambda i: (i, 0, 0)),
            pl.BlockSpec((1, 1, EB), lambda i: (i, 0, 0)),
            pl.BlockSpec((1, 1, EB), lambda i: (i, 0, 0)),
            full((24, 128)), full((1, 128)), full((128, 128)),
            full((128, 128)), full((128, 128)), full((1, 128)),
            full((128, 128)), full((1, 128)),
        ],
        out_specs=pl.BlockSpec((EB, 128), lambda i: (i, 0)),
        out_shape=jax.ShapeDtypeStruct((nblk * EB, 128), jnp.float32),
    )(w3, zr3, zc3, dW1p, db1, s_con, t_con, M, cvec, eW2, eb2)


def _sc_scatter(ef, row3, zeros):
    mesh = plsc.VectorSubcoreMesh(core_axis_name="c", subcore_axis_name="s")
    nchunk = row3.shape[1]
    epw = nchunk * CHUNK

    @functools.partial(
        pl.kernel,
        out_type=jax.ShapeDtypeStruct((NC, N, 128), jnp.float32),
        mesh=mesh,
        scratch_types=[pltpu.VMEM((nchunk, CHUNK), jnp.int32),
                       pltpu.VMEM((CHUNK, 128), jnp.float32),
                       pltpu.VMEM((CHUNK, 128), jnp.float32),
                       pltpu.VMEM_SHARED((N, 128), jnp.float32),
                       pltpu.SemaphoreType.DMA,
                       pltpu.SemaphoreType.DMA],
    )
    def k(ef_hbm, row_hbm, zero_hbm, out_hbm, idx_v, buf0, buf1, acc_sh,
          sem0, sem1):
        c = lax.axis_index("c")
        s = lax.axis_index("s")
        wid = s * NC + c
        base = wid * epw

        @pl.when(s < 10)
        def _zero():
            pltpu.sync_copy(zero_hbm, acc_sh.at[pl.ds(s * 1000, 1000)])

        pltpu.sync_copy(row_hbm.at[wid], idx_v)
        plsc.subcore_barrier()

        def fetch(j, buf, sem):
            off = pl.multiple_of(base + j * CHUNK, 8)
            pltpu.async_copy(ef_hbm.at[pl.ds(off, CHUNK)], buf, sem)

        def drain(buf, sem):
            pltpu.make_async_copy(ef_hbm.at[pl.ds(base, CHUNK)], buf, sem).wait()

        def scat(j, buf):
            pltpu.sync_copy(buf, acc_sh.at[idx_v.at[j]], add=True)

        fetch(0, buf0, sem0)

        def body(jj, _):
            j = 2 * jj
            fetch(j + 1, buf1, sem1)
            drain(buf0, sem0)
            scat(j, buf0)

            @pl.when(j + 2 < nchunk)
            def _pre():
                fetch(j + 2, buf0, sem0)

            drain(buf1, sem1)
            scat(j + 1, buf1)
            return ()

        lax.fori_loop(0, nchunk // 2, body, ())
        if nchunk % 2:
            drain(buf0, sem0)
            scat(nchunk - 1, buf0)
        plsc.subcore_barrier()

        @pl.when(s < 10)
        def _dump():
            pltpu.sync_copy(acc_sh.at[pl.ds(s * 1000, 1000)],
                            out_hbm.at[c, pl.ds(s * 1000, 1000)])

    return k(ef, row3, zeros)


def _tc_combine(p0, p1):
    def k(a_r, b_r, out_r):
        out_r[...] = (a_r[0] + a_r[1]) + (b_r[0] + b_r[1])

    spec = pl.BlockSpec((2, N // 10, 128), lambda i: (0, i, 0))
    return pl.pallas_call(
        k,
        grid=(10,),
        in_specs=[spec, spec],
        out_specs=pl.BlockSpec((N // 10, 128), lambda i: (i, 0)),
        out_shape=jax.ShapeDtypeStruct((N, 128), jnp.float32),
    )(p0, p1)


def _tc_sum2(a, b):
    def k(a_r, b_r, out_r):
        out_r[...] = a_r[...] + b_r[...]

    spec = pl.BlockSpec((N // 10, 128), lambda i: (i, 0))
    return pl.pallas_call(
        k,
        grid=(10,),
        in_specs=[spec, spec],
        out_specs=spec,
        out_shape=jax.ShapeDtypeStruct((N, 128), jnp.float32),
    )(a, b)


def kernel(z, edge_index, edge_weight, emb, dW1, db1, dW2, db2, sW1, sb1,
           sW2, sb2, tW1, tb1, tW2, tb2, eW1, eb1, eW2, eb2):
    z = z.astype(jnp.int32)
    row = edge_index[0].astype(jnp.int32)
    col = edge_index[1].astype(jnp.int32)

    zr, zc = _sc_gather_z(z, row, col)

    embp = jnp.zeros((128, 128), jnp.float32).at[:120].set(emb)
    s_con, t_con, M, cvec = _tc_tables(
        embp, sW1, sb1, sW2, sb2, tW1, tb1, tW2, tb2,
        eW1[0:128], eW1[128:256], eW1[256:384], dW2, db2, eb1)

    dW1p = jnp.zeros((24, 128), jnp.float32).at[:NB].set(dW1)
    zeros = jnp.zeros((1000, 128), jnp.float32)

    parts = []
    lo = 0
    for nblk in (31, 31, 31, 32):
        hi = lo + nblk * EB
        ef = _tc_edges(edge_weight[lo:hi].reshape(nblk, 1, EB),
                       zr[lo:hi].reshape(nblk, 1, EB),
                       zc[lo:hi].reshape(nblk, 1, EB),
                       dW1p, db1.reshape(1, 128), s_con, t_con, M, cvec,
                       eW2, eb2.reshape(1, 128))
        nch = (hi - lo) // NW // CHUNK
        parts.append(_sc_scatter(ef, row[lo:hi].reshape(NW, nch, CHUNK), zeros))
        lo = hi

    p01 = _tc_combine(parts[0], parts[1])
    p23 = _tc_combine(parts[2], parts[3])
    return _tc_sum2(p01, p23)


# speedup vs baseline: 9.1131x; 1.0385x over previous
import functools

import jax
import jax.numpy as jnp
from jax import lax
from jax.experimental import pallas as pl
from jax.experimental.pallas import tpu as pltpu
from jax.experimental.pallas import tpu_sc as plsc

N = 10000
E = 320000
NB = 20
CUTOFF = 5.0

NC = 2
NS = 16
NW = NC * NS
EPW = E // NW
CHUNK = 80
NCHUNK = EPW // CHUNK
ROWS_PER_TILE = N // NS

EB = 2560
NBLK = E // EB


def _silu(x):
    return x * jax.nn.sigmoid(x)


def _sc_gather_z(z, row, col):
    mesh = plsc.VectorSubcoreMesh(core_axis_name="c", subcore_axis_name="s")

    @functools.partial(
        pl.kernel,
        out_type=[jax.ShapeDtypeStruct((E,), jnp.int32),
                  jax.ShapeDtypeStruct((E,), jnp.int32)],
        mesh=mesh,
        scratch_types=[pltpu.VMEM((N,), jnp.int32),
                       pltpu.VMEM((EPW,), jnp.int32),
                       pltpu.VMEM((EPW,), jnp.int32)],
        compiler_params=pltpu.CompilerParams(needs_layout_passes=False),
    )
    def k(z_hbm, row_hbm, col_hbm, zr_hbm, zc_hbm, z_v, idx_v, out_v):
        wid = lax.axis_index("s") * NC + lax.axis_index("c")
        base = wid * EPW
        pltpu.sync_copy(z_hbm, z_v)

        def one(src_hbm, dst_hbm):
            pltpu.sync_copy(src_hbm.at[pl.ds(base, EPW)], idx_v)

            def body(i, _):
                iv = idx_v[pl.ds(i * 16, 16)]
                out_v[pl.ds(i * 16, 16)] = plsc.load_gather(z_v, [iv])
                return ()

            lax.fori_loop(0, EPW // 16, body, (), unroll=4)
            pltpu.sync_copy(out_v, dst_hbm.at[pl.ds(base, EPW)])

        one(row_hbm, zr_hbm)
        one(col_hbm, zc_hbm)

    return k(z, row, col)


def _tc_tables(embp, sW1, sb1, sW2, sb2, tW1, tb1, tW2, tb2,
               eW1_top, eW1_mid, eW1_bot, dW2, db2, eb1):
    def k(embp_r, sW1_r, sb1_r, sW2_r, sb2_r, tW1_r, tb1_r, tW2_r, tb2_r,
          eW1t_r, eW1m_r, eW1b_r, dW2_r, db2_r, eb1_r,
          sc_o, tc_o, m_o, c_o):
        f32 = jnp.float32
        emb_v = embp_r[...]
        s_all = _silu(jnp.dot(emb_v, sW1_r[...], preferred_element_type=f32) + sb1_r[...])
        s_all = jnp.dot(s_all, sW2_r[...], preferred_element_type=f32) + sb2_r[...]
        sc_o[...] = jnp.dot(s_all, eW1m_r[...], preferred_element_type=f32)
        t_all = _silu(jnp.dot(emb_v, tW1_r[...], preferred_element_type=f32) + tb1_r[...])
        t_all = jnp.dot(t_all, tW2_r[...], preferred_element_type=f32) + tb2_r[...]
        tc_o[...] = jnp.dot(t_all, eW1b_r[...], preferred_element_type=f32)
        m_o[...] = jnp.dot(dW2_r[...], eW1t_r[...], preferred_element_type=f32)
        c_o[...] = eb1_r[...] + jnp.dot(db2_r[...], eW1t_r[...], preferred_element_type=f32)

    return pl.pallas_call(
        k,
        out_shape=[jax.ShapeDtypeStruct((128, 128), jnp.float32),
                   jax.ShapeDtypeStruct((128, 128), jnp.float32),
                   jax.ShapeDtypeStruct((128, 128), jnp.float32),
                   jax.ShapeDtypeStruct((1, 128), jnp.float32)],
    )(embp, sW1, sb1.reshape(1, 128), sW2, sb2.reshape(1, 128),
      tW1, tb1.reshape(1, 128), tW2, tb2.reshape(1, 128),
      eW1_top, eW1_mid, eW1_bot, dW2, db2.reshape(1, 128), eb1.reshape(1, 128))


def _tc_edges(w3, zr3, zc3, dW1p, db1, s_con, t_con, M, cvec, eW2, eb2):
    def k(w_r, zr_r, zc_r, dW1_r, db1_r, sc_r, tc_r, m_r, c_r, eW2_r, eb2_r, out_r):
        f32 = jnp.float32
        r = w_r[0]
        n = lax.broadcasted_iota(jnp.int32, (24, EB), 0).astype(f32) + 1.0
        x = n * ((jnp.pi / CUTOFF) * r)
        k = jnp.round(x * 0.15915493667125702)
        xr = (x - k * 6.2831854820251465) - k * (-1.7484555314695172e-07)
        x2 = xr * xr
        p = jnp.float32(-2.069779872493349e-08)
        p = p * x2 + jnp.float32(2.708822857390436e-06)
        p = p * x2 + jnp.float32(-0.0001981762360091944)
        p = p * x2 + jnp.float32(0.008332791218600519)
        p = p * x2 + jnp.float32(-0.16666621064339257)
        p = p * x2 + jnp.float32(0.9999999376350313)
        sins = xr * p
        basis_t = (jnp.sqrt(2.0 / CUTOFF) / r) * sins
        hpre = lax.dot_general(basis_t, dW1_r[...],
                               (((0,), (0,)), ((), ())),
                               preferred_element_type=f32)
        h = _silu(hpre + db1_r[...])
        pre = jnp.dot(h, m_r[...], preferred_element_type=f32)
        lanes = lax.broadcasted_iota(jnp.int32, (EB, 128), 1)
        ohr = (lanes == zr_r[0, 0, :].reshape(EB, 1)).astype(f32)
        ohc = (lanes == zc_r[0, 0, :].reshape(EB, 1)).astype(f32)
        pre = pre + jnp.dot(ohr, sc_r[...], preferred_element_type=f32)
        pre = pre + jnp.dot(ohc, tc_r[...], preferred_element_type=f32)
        pre = _silu(pre + c_r[...])
        out_r[...] = jnp.dot(pre, eW2_r[...], preferred_element_type=f32) + eb2_r[...]

    nblk = w3.shape[0]
    full = lambda s: pl.BlockSpec(s, lambda i: (0,) * len(s))
    return pl.pallas_call(
        k,
        grid=(nblk,),
        in_specs=[
            pl.BlockSpec((1, 1, EB), lambda i: (i, 0, 0)),
            pl.BlockSpec((1, 1, EB), lambda i: (i, 0, 0)),
            pl.BlockSpec((1, 1, EB), lambda i: (i, 0, 0)),
            full((24, 128)), full((1, 128)), full((128, 128)),
            full((128, 128)), full((128, 128)), full((1, 128)),
            full((128, 128)), full((1, 128)),
        ],
        out_specs=pl.BlockSpec((EB, 128), lambda i: (i, 0)),
        out_shape=jax.ShapeDtypeStruct((nblk * EB, 128), jnp.float32),
    )(w3, zr3, zc3, dW1p, db1, s_con, t_con, M, cvec, eW2, eb2)


def _sc_scatter(ef, row3, init):
    mesh = plsc.VectorSubcoreMesh(core_axis_name="c", subcore_axis_name="s")
    nchunk = row3.shape[1]
    epw = nchunk * CHUNK

    @functools.partial(
        pl.kernel,
        out_type=jax.ShapeDtypeStruct((NC, N, 128), jnp.float32),
        mesh=mesh,
        scratch_types=[pltpu.VMEM((nchunk, CHUNK), jnp.int32),
                       pltpu.VMEM((CHUNK, 128), jnp.float32),
                       pltpu.VMEM((CHUNK, 128), jnp.float32),
                       pltpu.VMEM_SHARED((N, 128), jnp.float32),
                       pltpu.SemaphoreType.DMA,
                       pltpu.SemaphoreType.DMA],
    )
    def k(ef_hbm, row_hbm, init_hbm, out_hbm, idx_v, buf0, buf1, acc_sh,
          sem0, sem1):
        c = lax.axis_index("c")
        s = lax.axis_index("s")
        wid = s * NC + c
        base = wid * epw

        @pl.when(s < 10)
        def _init():
            pltpu.sync_copy(init_hbm.at[c, pl.ds(s * 1000, 1000)],
                            acc_sh.at[pl.ds(s * 1000, 1000)])

        pltpu.sync_copy(row_hbm.at[wid], idx_v)
        plsc.subcore_barrier()

        def fetch(j, buf, sem):
            off = pl.multiple_of(base + j * CHUNK, 8)
            pltpu.async_copy(ef_hbm.at[pl.ds(off, CHUNK)], buf, sem)

        def drain(buf, sem):
            pltpu.make_async_copy(ef_hbm.at[pl.ds(base, CHUNK)], buf, sem).wait()

        def scat(j, buf):
            pltpu.sync_copy(buf, acc_sh.at[idx_v.at[j]], add=True)

        fetch(0, buf0, sem0)

        def body(jj, _):
            j = 2 * jj
            fetch(j + 1, buf1, sem1)
            drain(buf0, sem0)
            scat(j, buf0)

            @pl.when(j + 2 < nchunk)
            def _pre():
                fetch(j + 2, buf0, sem0)

            drain(buf1, sem1)
            scat(j + 1, buf1)
            return ()

        lax.fori_loop(0, nchunk // 2, body, ())
        if nchunk % 2:
            drain(buf0, sem0)
            scat(nchunk - 1, buf0)
        plsc.subcore_barrier()

        @pl.when(s < 10)
        def _dump():
            pltpu.sync_copy(acc_sh.at[pl.ds(s * 1000, 1000)],
                            out_hbm.at[c, pl.ds(s * 1000, 1000)])

    return k(ef, row3, init)


def _tc_combine(p0, p1):
    def k(a_r, b_r, out_r):
        out_r[...] = (a_r[0] + a_r[1]) + (b_r[0] + b_r[1])

    spec = pl.BlockSpec((2, N // 10, 128), lambda i: (0, i, 0))
    return pl.pallas_call(
        k,
        grid=(10,),
        in_specs=[spec, spec],
        out_specs=pl.BlockSpec((N // 10, 128), lambda i: (i, 0)),
        out_shape=jax.ShapeDtypeStruct((N, 128), jnp.float32),
    )(p0, p1)


def _tc_sum2(a, b):
    def k(a_r, b_r, out_r):
        out_r[...] = a_r[...] + b_r[...]

    spec = pl.BlockSpec((N // 10, 128), lambda i: (i, 0))
    return pl.pallas_call(
        k,
        grid=(10,),
        in_specs=[spec, spec],
        out_specs=spec,
        out_shape=jax.ShapeDtypeStruct((N, 128), jnp.float32),
    )(a, b)


def kernel(z, edge_index, edge_weight, emb, dW1, db1, dW2, db2, sW1, sb1,
           sW2, sb2, tW1, tb1, tW2, tb2, eW1, eb1, eW2, eb2):
    z = z.astype(jnp.int32)
    row = edge_index[0].astype(jnp.int32)
    col = edge_index[1].astype(jnp.int32)

    zr, zc = _sc_gather_z(z, row, col)

    embp = jnp.zeros((128, 128), jnp.float32).at[:120].set(emb)
    s_con, t_con, M, cvec = _tc_tables(
        embp, sW1, sb1, sW2, sb2, tW1, tb1, tW2, tb2,
        eW1[0:128], eW1[128:256], eW1[256:384], dW2, db2, eb1)

    dW1p = jnp.zeros((24, 128), jnp.float32).at[:NB].set(dW1)
    part = jnp.zeros((NC, N, 128), jnp.float32)
    lo = 0
    for nblk in (31, 31, 31, 32):
        hi = lo + nblk * EB
        ef = _tc_edges(edge_weight[lo:hi].reshape(nblk, 1, EB),
                       zr[lo:hi].reshape(nblk, 1, EB),
                       zc[lo:hi].reshape(nblk, 1, EB),
                       dW1p, db1.reshape(1, 128), s_con, t_con, M, cvec,
                       eW2, eb2.reshape(1, 128))
        nch = (hi - lo) // NW // CHUNK
        part = _sc_scatter(ef, row[lo:hi].reshape(NW, nch, CHUNK), part)
        lo = hi

    return _tc_sum2(part[0], part[1])


# speedup vs baseline: 9.2571x; 1.0158x over previous
import functools

import jax
import jax.numpy as jnp
from jax import lax
from jax.experimental import pallas as pl
from jax.experimental.pallas import tpu as pltpu
from jax.experimental.pallas import tpu_sc as plsc

N = 10000
E = 320000
NB = 20
CUTOFF = 5.0

NC = 2
NS = 16
NW = NC * NS
EPW = E // NW
CHUNK = 80
NCHUNK = EPW // CHUNK
ROWS_PER_TILE = N // NS

EB = 2560
NBLK = E // EB


def _silu(x):
    return x * jax.nn.sigmoid(x)


def _sc_gather_z(z, row, col):
    mesh = plsc.VectorSubcoreMesh(core_axis_name="c", subcore_axis_name="s")

    @functools.partial(
        pl.kernel,
        out_type=[jax.ShapeDtypeStruct((E,), jnp.int32),
                  jax.ShapeDtypeStruct((E,), jnp.int32)],
        mesh=mesh,
        scratch_types=[pltpu.VMEM((N,), jnp.int32),
                       pltpu.VMEM((EPW,), jnp.int32),
                       pltpu.VMEM((EPW,), jnp.int32)],
        compiler_params=pltpu.CompilerParams(needs_layout_passes=False),
    )
    def k(z_hbm, row_hbm, col_hbm, zr_hbm, zc_hbm, z_v, idx_v, out_v):
        wid = lax.axis_index("s") * NC + lax.axis_index("c")
        base = wid * EPW
        pltpu.sync_copy(z_hbm, z_v)

        def one(src_hbm, dst_hbm):
            pltpu.sync_copy(src_hbm.at[pl.ds(base, EPW)], idx_v)

            def body(i, _):
                iv = idx_v[pl.ds(i * 16, 16)]
                out_v[pl.ds(i * 16, 16)] = plsc.load_gather(z_v, [iv])
                return ()

            lax.fori_loop(0, EPW // 16, body, (), unroll=4)
            pltpu.sync_copy(out_v, dst_hbm.at[pl.ds(base, EPW)])

        one(row_hbm, zr_hbm)
        one(col_hbm, zc_hbm)

    return k(z, row, col)


def _tc_tables(embp, sW1, sb1, sW2, sb2, tW1, tb1, tW2, tb2,
               eW1_top, eW1_mid, eW1_bot, dW2, db2, eb1):
    def k(embp_r, sW1_r, sb1_r, sW2_r, sb2_r, tW1_r, tb1_r, tW2_r, tb2_r,
          eW1t_r, eW1m_r, eW1b_r, dW2_r, db2_r, eb1_r,
          sc_o, tc_o, m_o, c_o):
        f32 = jnp.float32
        emb_v = embp_r[...]
        s_all = _silu(jnp.dot(emb_v, sW1_r[...], preferred_element_type=f32) + sb1_r[...])
        s_all = jnp.dot(s_all, sW2_r[...], preferred_element_type=f32) + sb2_r[...]
        sc_o[...] = jnp.dot(s_all, eW1m_r[...], preferred_element_type=f32)
        t_all = _silu(jnp.dot(emb_v, tW1_r[...], preferred_element_type=f32) + tb1_r[...])
        t_all = jnp.dot(t_all, tW2_r[...], preferred_element_type=f32) + tb2_r[...]
        tc_o[...] = jnp.dot(t_all, eW1b_r[...], preferred_element_type=f32)
        m_o[...] = jnp.dot(dW2_r[...], eW1t_r[...], preferred_element_type=f32)
        c_o[...] = eb1_r[...] + jnp.dot(db2_r[...], eW1t_r[...], preferred_element_type=f32)

    return pl.pallas_call(
        k,
        out_shape=[jax.ShapeDtypeStruct((128, 128), jnp.float32),
                   jax.ShapeDtypeStruct((128, 128), jnp.float32),
                   jax.ShapeDtypeStruct((128, 128), jnp.float32),
                   jax.ShapeDtypeStruct((1, 128), jnp.float32)],
    )(embp, sW1, sb1.reshape(1, 128), sW2, sb2.reshape(1, 128),
      tW1, tb1.reshape(1, 128), tW2, tb2.reshape(1, 128),
      eW1_top, eW1_mid, eW1_bot, dW2, db2.reshape(1, 128), eb1.reshape(1, 128))


def _tc_edges(w3, zr3, zc3, dW1p, db1, s_con, t_con, M, cvec, eW2, eb2):
    def k(w_r, zr_r, zc_r, dW1_r, db1_r, sc_r, tc_r, m_r, c_r, eW2_r, eb2_r, out_r):
        f32 = jnp.float32
        r = w_r[0]
        n = lax.broadcasted_iota(jnp.int32, (24, EB), 0).astype(f32) + 1.0
        x = n * ((jnp.pi / CUTOFF) * r)
        k = jnp.round(x * 0.15915493667125702)
        xr = (x - k * 6.2831854820251465) - k * (-1.7484555314695172e-07)
        x2 = xr * xr
        p = jnp.float32(-2.069779872493349e-08)
        p = p * x2 + jnp.float32(2.708822857390436e-06)
        p = p * x2 + jnp.float32(-0.0001981762360091944)
        p = p * x2 + jnp.float32(0.008332791218600519)
        p = p * x2 + jnp.float32(-0.16666621064339257)
        p = p * x2 + jnp.float32(0.9999999376350313)
        sins = xr * p
        basis_t = (jnp.sqrt(2.0 / CUTOFF) / r) * sins
        hpre = lax.dot_general(basis_t, dW1_r[...],
                               (((0,), (0,)), ((), ())),
                               preferred_element_type=f32)
        h = _silu(hpre + db1_r[...])
        pre = jnp.dot(h, m_r[...], preferred_element_type=f32)
        lanes = lax.broadcasted_iota(jnp.int32, (EB, 128), 1)
        ohr = (lanes == zr_r[0, 0, :].reshape(EB, 1)).astype(f32)
        ohc = (lanes == zc_r[0, 0, :].reshape(EB, 1)).astype(f32)
        pre = pre + jnp.dot(ohr, sc_r[...], preferred_element_type=f32)
        pre = pre + jnp.dot(ohc, tc_r[...], preferred_element_type=f32)
        pre = _silu(pre + c_r[...])
        out_r[...] = jnp.dot(pre, eW2_r[...], preferred_element_type=f32) + eb2_r[...]

    nblk = w3.shape[0]
    full = lambda s: pl.BlockSpec(s, lambda i: (0,) * len(s))
    return pl.pallas_call(
        k,
        grid=(nblk,),
        in_specs=[
            pl.BlockSpec((1, 1, EB), lambda i: (i, 0, 0)),
            pl.BlockSpec((1, 1, EB), lambda i: (i, 0, 0)),
            pl.BlockSpec((1, 1, EB), lambda i: (i, 0, 0)),
            full((24, 128)), full((1, 128)), full((128, 128)),
            full((128, 128)), full((128, 128)), full((1, 128)),
            full((128, 128)), full((1, 128)),
        ],
        out_specs=pl.BlockSpec((EB, 128), lambda i: (i, 0)),
        out_shape=jax.ShapeDtypeStruct((nblk * EB, 128), jnp.float32),
    )(w3, zr3, zc3, dW1p, db1, s_con, t_con, M, cvec, eW2, eb2)


def _sc_scatter(ef, row3, init):
    mesh = plsc.VectorSubcoreMesh(core_axis_name="c", subcore_axis_name="s")
    nchunk, chunk = row3.shape[1], row3.shape[2]
    epw = nchunk * chunk

    @functools.partial(
        pl.kernel,
        out_type=jax.ShapeDtypeStruct((NC, N, 128), jnp.float32),
        mesh=mesh,
        scratch_types=[pltpu.VMEM((nchunk, chunk), jnp.int32),
                       pltpu.VMEM((chunk, 128), jnp.float32),
                       pltpu.VMEM((chunk, 128), jnp.float32),
                       pltpu.VMEM_SHARED((N, 128), jnp.float32),
                       pltpu.SemaphoreType.DMA,
                       pltpu.SemaphoreType.DMA],
    )
    def k(ef_hbm, row_hbm, init_hbm, out_hbm, idx_v, buf0, buf1, acc_sh,
          sem0, sem1):
        c = lax.axis_index("c")
        s = lax.axis_index("s")
        wid = s * NC + c
        base = wid * epw

        @pl.when(s < 10)
        def _init():
            pltpu.sync_copy(init_hbm.at[c, pl.ds(s * 1000, 1000)],
                            acc_sh.at[pl.ds(s * 1000, 1000)])

        pltpu.sync_copy(row_hbm.at[wid], idx_v)
        plsc.subcore_barrier()

        def fetch(j, buf, sem):
            off = pl.multiple_of(base + j * chunk, 8)
            pltpu.async_copy(ef_hbm.at[pl.ds(off, chunk)], buf, sem)

        def drain(buf, sem):
            pltpu.make_async_copy(ef_hbm.at[pl.ds(base, chunk)], buf, sem).wait()

        def scat(j, buf):
            pltpu.sync_copy(buf, acc_sh.at[idx_v.at[j]], add=True)

        fetch(0, buf0, sem0)

        def body(jj, _):
            j = 2 * jj
            fetch(j + 1, buf1, sem1)
            drain(buf0, sem0)
            scat(j, buf0)

            @pl.when(j + 2 < nchunk)
            def _pre():
                fetch(j + 2, buf0, sem0)

            drain(buf1, sem1)
            scat(j + 1, buf1)
            return ()

        lax.fori_loop(0, nchunk // 2, body, ())
        if nchunk % 2:
            drain(buf0, sem0)
            scat(nchunk - 1, buf0)
        plsc.subcore_barrier()

        @pl.when(s < 10)
        def _dump():
            pltpu.sync_copy(acc_sh.at[pl.ds(s * 1000, 1000)],
                            out_hbm.at[c, pl.ds(s * 1000, 1000)])

    return k(ef, row3, init)


def _tc_combine(p0, p1):
    def k(a_r, b_r, out_r):
        out_r[...] = (a_r[0] + a_r[1]) + (b_r[0] + b_r[1])

    spec = pl.BlockSpec((2, N // 10, 128), lambda i: (0, i, 0))
    return pl.pallas_call(
        k,
        grid=(10,),
        in_specs=[spec, spec],
        out_specs=pl.BlockSpec((N // 10, 128), lambda i: (i, 0)),
        out_shape=jax.ShapeDtypeStruct((N, 128), jnp.float32),
    )(p0, p1)


def _tc_sum2(a, b):
    def k(a_r, b_r, out_r):
        out_r[...] = a_r[...] + b_r[...]

    spec = pl.BlockSpec((N // 10, 128), lambda i: (i, 0))
    return pl.pallas_call(
        k,
        grid=(10,),
        in_specs=[spec, spec],
        out_specs=spec,
        out_shape=jax.ShapeDtypeStruct((N, 128), jnp.float32),
    )(a, b)


def kernel(z, edge_index, edge_weight, emb, dW1, db1, dW2, db2, sW1, sb1,
           sW2, sb2, tW1, tb1, tW2, tb2, eW1, eb1, eW2, eb2):
    z = z.astype(jnp.int32)
    row = edge_index[0].astype(jnp.int32)
    col = edge_index[1].astype(jnp.int32)

    zr, zc = _sc_gather_z(z, row, col)

    embp = jnp.zeros((128, 128), jnp.float32).at[:120].set(emb)
    s_con, t_con, M, cvec = _tc_tables(
        embp, sW1, sb1, sW2, sb2, tW1, tb1, tW2, tb2,
        eW1[0:128], eW1[128:256], eW1[256:384], dW2, db2, eb1)

    dW1p = jnp.zeros((24, 128), jnp.float32).at[:NB].set(dW1)
    part = jnp.zeros((NC, N, 128), jnp.float32)
    lo = 0
    for nblk, chunk in ((32, 128), (32, 128), (32, 128), (29, 80)):
        hi = lo + nblk * EB
        ef = _tc_edges(edge_weight[lo:hi].reshape(nblk, 1, EB),
                       zr[lo:hi].reshape(nblk, 1, EB),
                       zc[lo:hi].reshape(nblk, 1, EB),
                       dW1p, db1.reshape(1, 128), s_con, t_con, M, cvec,
                       eW2, eb2.reshape(1, 128))
        nch = (hi - lo) // NW // chunk
        part = _sc_scatter(ef, row[lo:hi].reshape(NW, nch, chunk), part)
        lo = hi

    return _tc_sum2(part[0], part[1])


# speedup vs baseline: 9.3807x; 1.0133x over previous
import functools

import jax
import jax.numpy as jnp
from jax import lax
from jax.experimental import pallas as pl
from jax.experimental.pallas import tpu as pltpu
from jax.experimental.pallas import tpu_sc as plsc

N = 10000
E = 320000
NB = 20
CUTOFF = 5.0

NC = 2
NS = 16
NW = NC * NS
EPW = E // NW
CHUNK = 80
NCHUNK = EPW // CHUNK
ROWS_PER_TILE = N // NS

EB = 2560
NBLK = E // EB


def _silu(x):
    return x * jax.nn.sigmoid(x)


def _sc_gather_z(z, row, col):
    mesh = plsc.VectorSubcoreMesh(core_axis_name="c", subcore_axis_name="s")
    seg = row.shape[0]
    epw = seg // NW

    @functools.partial(
        pl.kernel,
        out_type=[jax.ShapeDtypeStruct((seg,), jnp.int32),
                  jax.ShapeDtypeStruct((seg,), jnp.int32)],
        mesh=mesh,
        scratch_types=[pltpu.VMEM((N,), jnp.int32),
                       pltpu.VMEM((epw,), jnp.int32),
                       pltpu.VMEM((epw,), jnp.int32)],
        compiler_params=pltpu.CompilerParams(needs_layout_passes=False),
    )
    def k(z_hbm, row_hbm, col_hbm, zr_hbm, zc_hbm, z_v, idx_v, out_v):
        wid = lax.axis_index("s") * NC + lax.axis_index("c")
        base = wid * epw
        pltpu.sync_copy(z_hbm, z_v)

        def one(src_hbm, dst_hbm):
            pltpu.sync_copy(src_hbm.at[pl.ds(base, epw)], idx_v)

            def body(i, _):
                iv = idx_v[pl.ds(i * 16, 16)]
                out_v[pl.ds(i * 16, 16)] = plsc.load_gather(z_v, [iv])
                return ()

            lax.fori_loop(0, epw // 16, body, (), unroll=4)
            pltpu.sync_copy(out_v, dst_hbm.at[pl.ds(base, epw)])

        one(row_hbm, zr_hbm)
        one(col_hbm, zc_hbm)

    return k(z, row, col)


def _tc_tables(embp, sW1, sb1, sW2, sb2, tW1, tb1, tW2, tb2,
               eW1_top, eW1_mid, eW1_bot, dW2, db2, eb1):
    def k(embp_r, sW1_r, sb1_r, sW2_r, sb2_r, tW1_r, tb1_r, tW2_r, tb2_r,
          eW1t_r, eW1m_r, eW1b_r, dW2_r, db2_r, eb1_r,
          sc_o, tc_o, m_o, c_o):
        f32 = jnp.float32
        emb_v = embp_r[...]
        s_all = _silu(jnp.dot(emb_v, sW1_r[...], preferred_element_type=f32) + sb1_r[...])
        s_all = jnp.dot(s_all, sW2_r[...], preferred_element_type=f32) + sb2_r[...]
        sc_o[...] = jnp.dot(s_all, eW1m_r[...], preferred_element_type=f32)
        t_all = _silu(jnp.dot(emb_v, tW1_r[...], preferred_element_type=f32) + tb1_r[...])
        t_all = jnp.dot(t_all, tW2_r[...], preferred_element_type=f32) + tb2_r[...]
        tc_o[...] = jnp.dot(t_all, eW1b_r[...], preferred_element_type=f32)
        m_o[...] = jnp.dot(dW2_r[...], eW1t_r[...], preferred_element_type=f32)
        c_o[...] = eb1_r[...] + jnp.dot(db2_r[...], eW1t_r[...], preferred_element_type=f32)

    return pl.pallas_call(
        k,
        out_shape=[jax.ShapeDtypeStruct((128, 128), jnp.float32),
                   jax.ShapeDtypeStruct((128, 128), jnp.float32),
                   jax.ShapeDtypeStruct((128, 128), jnp.float32),
                   jax.ShapeDtypeStruct((1, 128), jnp.float32)],
    )(embp, sW1, sb1.reshape(1, 128), sW2, sb2.reshape(1, 128),
      tW1, tb1.reshape(1, 128), tW2, tb2.reshape(1, 128),
      eW1_top, eW1_mid, eW1_bot, dW2, db2.reshape(1, 128), eb1.reshape(1, 128))


def _tc_edges(w3, zr3, zc3, dW1p, db1, s_con, t_con, M, cvec, eW2, eb2):
    def k(w_r, zr_r, zc_r, dW1_r, db1_r, sc_r, tc_r, m_r, c_r, eW2_r, eb2_r, out_r):
        f32 = jnp.float32
        r = w_r[0]
        n = lax.broadcasted_iota(jnp.int32, (24, EB), 0).astype(f32) + 1.0
        x = n * ((jnp.pi / CUTOFF) * r)
        k = jnp.round(x * 0.15915493667125702)
        xr = (x - k * 6.2831854820251465) - k * (-1.7484555314695172e-07)
        x2 = xr * xr
        p = jnp.float32(-2.069779872493349e-08)
        p = p * x2 + jnp.float32(2.708822857390436e-06)
        p = p * x2 + jnp.float32(-0.0001981762360091944)
        p = p * x2 + jnp.float32(0.008332791218600519)
        p = p * x2 + jnp.float32(-0.16666621064339257)
        p = p * x2 + jnp.float32(0.9999999376350313)
        sins = xr * p
        basis_t = (jnp.sqrt(2.0 / CUTOFF) / r) * sins
        hpre = lax.dot_general(basis_t, dW1_r[...],
                               (((0,), (0,)), ((), ())),
                               preferred_element_type=f32)
        h = _silu(hpre + db1_r[...])
        pre = jnp.dot(h, m_r[...], preferred_element_type=f32)
        lanes = lax.broadcasted_iota(jnp.int32, (EB, 128), 1)
        ohr = (lanes == zr_r[0, 0, :].reshape(EB, 1)).astype(f32)
        ohc = (lanes == zc_r[0, 0, :].reshape(EB, 1)).astype(f32)
        pre = pre + jnp.dot(ohr, sc_r[...], preferred_element_type=f32)
        pre = pre + jnp.dot(ohc, tc_r[...], preferred_element_type=f32)
        pre = _silu(pre + c_r[...])
        out_r[...] = jnp.dot(pre, eW2_r[...], preferred_element_type=f32) + eb2_r[...]

    nblk = w3.shape[0]
    full = lambda s: pl.BlockSpec(s, lambda i: (0,) * len(s))
    return pl.pallas_call(
        k,
        grid=(nblk,),
        in_specs=[
            pl.BlockSpec((1, 1, EB), lambda i: (i, 0, 0)),
            pl.BlockSpec((1, 1, EB), lambda i: (i, 0, 0)),
            pl.BlockSpec((1, 1, EB), lambda i: (i, 0, 0)),
            full((24, 128)), full((1, 128)), full((128, 128)),
            full((128, 128)), full((128, 128)), full((1, 128)),
            full((128, 128)), full((1, 128)),
        ],
        out_specs=pl.BlockSpec((EB, 128), lambda i: (i, 0)),
        out_shape=jax.ShapeDtypeStruct((nblk * EB, 128), jnp.float32),
    )(w3, zr3, zc3, dW1p, db1, s_con, t_con, M, cvec, eW2, eb2)


def _sc_scatter(ef, row3, init):
    mesh = plsc.VectorSubcoreMesh(core_axis_name="c", subcore_axis_name="s")
    nchunk, chunk = row3.shape[1], row3.shape[2]
    epw = nchunk * chunk

    @functools.partial(
        pl.kernel,
        out_type=jax.ShapeDtypeStruct((NC, N, 128), jnp.float32),
        mesh=mesh,
        scratch_types=[pltpu.VMEM((nchunk, chunk), jnp.int32),
                       pltpu.VMEM((chunk, 128), jnp.float32),
                       pltpu.VMEM((chunk, 128), jnp.float32),
                       pltpu.VMEM_SHARED((N, 128), jnp.float32),
                       pltpu.SemaphoreType.DMA,
                       pltpu.SemaphoreType.DMA],
    )
    def k(ef_hbm, row_hbm, init_hbm, out_hbm, idx_v, buf0, buf1, acc_sh,
          sem0, sem1):
        c = lax.axis_index("c")
        s = lax.axis_index("s")
        wid = s * NC + c
        base = wid * epw

        @pl.when(s < 10)
        def _init():
            pltpu.sync_copy(init_hbm.at[c, pl.ds(s * 1000, 1000)],
                            acc_sh.at[pl.ds(s * 1000, 1000)])

        pltpu.sync_copy(row_hbm.at[wid], idx_v)
        plsc.subcore_barrier()

        def fetch(j, buf, sem):
            off = pl.multiple_of(base + j * chunk, 8)
            pltpu.async_copy(ef_hbm.at[pl.ds(off, chunk)], buf, sem)

        def drain(buf, sem):
            pltpu.make_async_copy(ef_hbm.at[pl.ds(base, chunk)], buf, sem).wait()

        def scat(j, buf):
            pltpu.sync_copy(buf, acc_sh.at[idx_v.at[j]], add=True)

        fetch(0, buf0, sem0)

        def body(jj, _):
            j = 2 * jj
            fetch(j + 1, buf1, sem1)
            drain(buf0, sem0)
            scat(j, buf0)

            @pl.when(j + 2 < nchunk)
            def _pre():
                fetch(j + 2, buf0, sem0)

            drain(buf1, sem1)
            scat(j + 1, buf1)
            return ()

        lax.fori_loop(0, nchunk // 2, body, ())
        if nchunk % 2:
            drain(buf0, sem0)
            scat(nchunk - 1, buf0)
        plsc.subcore_barrier()

        @pl.when(s < 10)
        def _dump():
            pltpu.sync_copy(acc_sh.at[pl.ds(s * 1000, 1000)],
                            out_hbm.at[c, pl.ds(s * 1000, 1000)])

    return k(ef, row3, init)


def _tc_combine(p0, p1):
    def k(a_r, b_r, out_r):
        out_r[...] = (a_r[0] + a_r[1]) + (b_r[0] + b_r[1])

    spec = pl.BlockSpec((2, N // 10, 128), lambda i: (0, i, 0))
    return pl.pallas_call(
        k,
        grid=(10,),
        in_specs=[spec, spec],
        out_specs=pl.BlockSpec((N // 10, 128), lambda i: (i, 0)),
        out_shape=jax.ShapeDtypeStruct((N, 128), jnp.float32),
    )(p0, p1)


def _tc_sum2(a, b):
    def k(a_r, b_r, out_r):
        out_r[...] = a_r[...] + b_r[...]

    spec = pl.BlockSpec((N // 10, 128), lambda i: (i, 0))
    return pl.pallas_call(
        k,
        grid=(10,),
        in_specs=[spec, spec],
        out_specs=spec,
        out_shape=jax.ShapeDtypeStruct((N, 128), jnp.float32),
    )(a, b)


def kernel(z, edge_index, edge_weight, emb, dW1, db1, dW2, db2, sW1, sb1,
           sW2, sb2, tW1, tb1, tW2, tb2, eW1, eb1, eW2, eb2):
    z = z.astype(jnp.int32)
    row = edge_index[0].astype(jnp.int32)
    col = edge_index[1].astype(jnp.int32)

    s0 = 16 * EB
    zr0, zc0 = _sc_gather_z(z, row[:s0], col[:s0])
    zr1, zc1 = _sc_gather_z(z, row[s0:], col[s0:])

    embp = jnp.zeros((128, 128), jnp.float32).at[:120].set(emb)
    s_con, t_con, M, cvec = _tc_tables(
        embp, sW1, sb1, sW2, sb2, tW1, tb1, tW2, tb2,
        eW1[0:128], eW1[128:256], eW1[256:384], dW2, db2, eb1)

    dW1p = jnp.zeros((24, 128), jnp.float32).at[:NB].set(dW1)
    part = jnp.zeros((NC, N, 128), jnp.float32)
    lo = 0
    for nblk, chunk in ((16, 128), (32, 128), (32, 128), (32, 128), (13, 80)):
        hi = lo + nblk * EB
        if hi <= s0:
            zrs, zcs = zr0[lo:hi], zc0[lo:hi]
        else:
            zrs, zcs = zr1[lo - s0:hi - s0], zc1[lo - s0:hi - s0]
        ef = _tc_edges(edge_weight[lo:hi].reshape(nblk, 1, EB),
                       zrs.reshape(nblk, 1, EB), zcs.reshape(nblk, 1, EB),
                       dW1p, db1.reshape(1, 128), s_con, t_con, M, cvec,
                       eW2, eb2.reshape(1, 128))
        nch = (hi - lo) // NW // chunk
        part = _sc_scatter(ef, row[lo:hi].reshape(NW, nch, chunk), part)
        lo = hi

    return _tc_sum2(part[0], part[1])
